# baseline (speedup 1.0000x reference)
import dataclasses
import functools

import jax
import jax.numpy as jnp
from jax import lax
from jax.experimental import pallas as pl
from jax.experimental.pallas import tpu as pltpu
from jax.experimental.pallas import tpu_sc as plsc

N = 50000
E = 800000
NODE_DIM = 12
EDGE_DIM = 4
HIDDEN = 128
HEADS = 4
C = 32
LAYERS = 3
NGRAPH = 64

NB = 1000
EB = 2048
E_PAD = 802816
R_TOT = E_PAD // 128
MAC = 4
MAC2 = 8
NSUB = 16
NWORK = 32
N_PAD = 50176
ROWS_PER_SUB = N_PAD // NSUB
ZR = 112

_f32 = jnp.float32


def _gelu(x):
    return 0.5 * x * (1.0 + lax.erf(x * (2.0 ** -0.5)))


def _ln_rows(h, g, b):
    mu = jnp.mean(h, axis=-1, keepdims=True)
    v = jnp.mean((h - mu) ** 2, axis=-1, keepdims=True)
    return (h - mu) / jnp.sqrt(v + 1e-5) * g + b


def _node_enc(x, W, b, g, beta):
    def body(x_ref, w_ref, b_ref, g_ref, be_ref, o_ref):
        h = jnp.dot(x_ref[...], w_ref[...], preferred_element_type=_f32)
        h = h + b_ref[...]
        o_ref[...] = _gelu(_ln_rows(h, g_ref[...], be_ref[...]))

    return pl.pallas_call(
        body,
        grid=(N // NB,),
        in_specs=[
            pl.BlockSpec((NB, NODE_DIM), lambda i: (i, 0)),
            pl.BlockSpec((NODE_DIM, HIDDEN), lambda i: (0, 0)),
            pl.BlockSpec((1, HIDDEN), lambda i: (0, 0)),
            pl.BlockSpec((1, HIDDEN), lambda i: (0, 0)),
            pl.BlockSpec((1, HIDDEN), lambda i: (0, 0)),
        ],
        out_specs=pl.BlockSpec((NB, HIDDEN), lambda i: (i, 0)),
        out_shape=jax.ShapeDtypeStruct((N, HIDDEN), _f32),
    )(x, W, b, g, beta)


def _edge_enc(edge_attr, eeW, eeb, V):
    def body(ea_ref, w_ref, b_ref, v0_ref, v1_ref, v2_ref, o0, o1, o2):
        t = jnp.dot(ea_ref[...], w_ref[...], preferred_element_type=_f32)
        t = _gelu(t + b_ref[...])
        o0[...] = jnp.dot(t, v0_ref[...], preferred_element_type=_f32)
        o1[...] = jnp.dot(t, v1_ref[...], preferred_element_type=_f32)
        o2[...] = jnp.dot(t, v2_ref[...], preferred_element_type=_f32)

    os = jax.ShapeDtypeStruct((E_PAD, 16), _f32)
    return pl.pallas_call(
        body,
        grid=(E_PAD // EB,),
        in_specs=[
            pl.BlockSpec((EB, EDGE_DIM), lambda i: (i, 0)),
            pl.BlockSpec((EDGE_DIM, HIDDEN // 2), lambda i: (0, 0)),
            pl.BlockSpec((1, HIDDEN // 2), lambda i: (0, 0)),
            pl.BlockSpec((HIDDEN // 2, 16), lambda i: (0, 0)),
            pl.BlockSpec((HIDDEN // 2, 16), lambda i: (0, 0)),
            pl.BlockSpec((HIDDEN // 2, 16), lambda i: (0, 0)),
        ],
        out_specs=[pl.BlockSpec((EB, 16), lambda i: (i, 0))] * 3,
        out_shape=[os, os, os],
    )(edge_attr, eeW, eeb, V[0], V[1], V[2])


def _layer_proj(h, W, aM, dM):
    def body(h_ref, w_ref, a_ref, d_ref, xh_ref, as_ref, ad_ref):
        xh = jnp.dot(h_ref[...], w_ref[...], preferred_element_type=_f32)
        xh_ref[...] = xh
        as_ref[...] = jnp.dot(xh, a_ref[...], preferred_element_type=_f32)
        ad_ref[...] = jnp.dot(xh, d_ref[...], preferred_element_type=_f32)

    return pl.pallas_call(
        body,
        grid=(N // NB,),
        in_specs=[
            pl.BlockSpec((NB, HIDDEN), lambda i: (i, 0)),
            pl.BlockSpec((HIDDEN, HIDDEN), lambda i: (0, 0)),
            pl.BlockSpec((HIDDEN, 16), lambda i: (0, 0)),
            pl.BlockSpec((HIDDEN, 16), lambda i: (0, 0)),
        ],
        out_specs=[
            pl.BlockSpec((NB, HIDDEN), lambda i: (i, 0)),
            pl.BlockSpec((NB, 16), lambda i: (i, 0)),
            pl.BlockSpec((NB, 16), lambda i: (i, 0)),
        ],
        out_shape=[
            jax.ShapeDtypeStruct((N, HIDDEN), _f32),
            jax.ShapeDtypeStruct((N, 16), _f32),
            jax.ShapeDtypeStruct((N, 16), _f32),
        ],
    )(h, W, aM, dM)


def _layer_norm_res(h, out4, denp, R4, gb, lg, lb):
    def body(h_ref, o_ref, dp_ref, r_ref, gb_ref, g_ref, b_ref, ho_ref):
        den = dp_ref[0, :, 0:4] + dp_ref[1, :, 0:4]
        rec = 1.0 / (den + 1e-16)
        drep = jnp.dot(rec, r_ref[...], preferred_element_type=_f32)
        cat = jnp.concatenate([o_ref[q] for q in range(8)], axis=-1)
        val = h_ref[...] + cat * drep + gb_ref[...]
        ho_ref[...] = _ln_rows(val, g_ref[...], b_ref[...])

    return pl.pallas_call(
        body,
        grid=(N // NB,),
        in_specs=[
            pl.BlockSpec((NB, HIDDEN), lambda i: (i, 0)),
            pl.BlockSpec((2 * HEADS, NB, 16), lambda i: (0, i, 0)),
            pl.BlockSpec((2, NB, 16), lambda i: (0, i, 0)),
            pl.BlockSpec((HEADS, HIDDEN), lambda i: (0, 0)),
            pl.BlockSpec((1, HIDDEN), lambda i: (0, 0)),
            pl.BlockSpec((1, HIDDEN), lambda i: (0, 0)),
            pl.BlockSpec((1, HIDDEN), lambda i: (0, 0)),
        ],
        out_specs=pl.BlockSpec((NB, HIDDEN), lambda i: (i, 0)),
        out_shape=jax.ShapeDtypeStruct((N, HIDDEN), _f32),
    )(h, out4, denp, R4, gb, lg, lb)


def _pool(h, batch3):
    def body(h_ref, b_ref, o_ref, s_ref, m_ref, c_ref):
        ones = jnp.ones((NB, HIDDEN), _f32)
        i = pl.program_id(0)

        @pl.when(i == 0)
        def _():
            s_ref[...] = jnp.zeros((NGRAPH, HIDDEN), _f32)
            c_ref[...] = jnp.zeros((NGRAPH, HIDDEN), _f32)
            m_ref[...] = jnp.full((NGRAPH, HIDDEN), -jnp.inf, _f32)

        hb = h_ref[...]
        b = b_ref[0, 0, :]
        oh = (b[:, None] == lax.broadcasted_iota(
            jnp.int32, (NB, NGRAPH), 1)).astype(_f32)
        dn = (((0,), (0,)), ((), ()))
        s_ref[...] += lax.dot_general(oh, hb, dn,
                                      preferred_element_type=_f32)
        c_ref[...] += lax.dot_general(oh, ones, dn,
                                      preferred_element_type=_f32)
        mrows = [
            jnp.max(jnp.where(b[:, None] == g, hb, -jnp.inf), axis=0,
                    keepdims=True)
            for g in range(NGRAPH)
        ]
        m_ref[...] = jnp.maximum(m_ref[...], jnp.concatenate(mrows, axis=0))

        cnt = c_ref[...]
        mean = s_ref[...] / jnp.maximum(cnt, 1.0)
        mx = jnp.where(cnt > 0.0, m_ref[...], 0.0)
        o_ref[...] = jnp.concatenate([mean, mx], axis=-1)

    return pl.pallas_call(
        body,
        grid=(N // NB,),
        in_specs=[
            pl.BlockSpec((NB, HIDDEN), lambda i: (i, 0)),
            pl.BlockSpec((1, 1, NB), lambda i: (i, 0, 0)),
        ],
        out_specs=pl.BlockSpec((NGRAPH, 2 * HIDDEN), lambda i: (0, 0)),
        out_shape=jax.ShapeDtypeStruct((NGRAPH, 2 * HIDDEN), _f32),
        scratch_shapes=[
            pltpu.VMEM((NGRAPH, HIDDEN), _f32),
            pltpu.VMEM((NGRAPH, HIDDEN), _f32),
            pltpu.VMEM((NGRAPH, HIDDEN), _f32),
        ],
    )(h, batch3)


def _head_mlp(g, f1W, f1b, f2W, f2b, c1W, c1b, c2r, c2b):
    def body(g_ref, w1, b1, w2, b2, w3, b3, w4, b4, o_ref):
        x = _gelu(jnp.dot(g_ref[...], w1[...],
                          preferred_element_type=_f32) + b1[...])
        x = _gelu(jnp.dot(x, w2[...],
                          preferred_element_type=_f32) + b2[...])
        x = _gelu(jnp.dot(x, w3[...],
                          preferred_element_type=_f32) + b3[...])
        logits = jnp.sum(x * w4[...], axis=1, keepdims=True) + b4[...]
        o_ref[...] = 1.0 / (1.0 + jnp.exp(-logits))

    full = lambda s: pl.BlockSpec(s, lambda: tuple(0 for _ in s))
    return pl.pallas_call(
        body,
        in_specs=[full((NGRAPH, 2 * HIDDEN)),
                  full((2 * HIDDEN, HIDDEN)), full((1, HIDDEN)),
                  full((HIDDEN, HIDDEN // 2)), full((1, HIDDEN // 2)),
                  full((HIDDEN // 2, 64)), full((1, 64)),
                  full((1, 64)), full((1, 1))],
        out_specs=full((NGRAPH, 1)),
        out_shape=jax.ShapeDtypeStruct((NGRAPH, 1), _f32),
    )(g, f1W, f1b, f2W, f2b, c1W, c1b, c2r, c2b)


_MESH = plsc.VectorSubcoreMesh(core_axis_name="c", subcore_axis_name="s")
_SC_PARAMS = pltpu.CompilerParams()
if "needs_layout_passes" in pltpu.CompilerParams.__dataclass_fields__:
    _SC_PARAMS = dataclasses.replace(_SC_PARAMS, needs_layout_passes=False)
if "use_tc_tiling_on_sc" in pltpu.CompilerParams.__dataclass_fields__:
    _SC_PARAMS = dataclasses.replace(_SC_PARAMS, use_tc_tiling_on_sc=False)

RPW1 = R_TOT // NWORK
RPS2 = R_TOT // NSUB


def _sc_pass1(src2, dst2, asrc16, adst16, ae16):
    @functools.partial(
        pl.kernel,
        mesh=_MESH,
        compiler_params=_SC_PARAMS,
        out_type=[jax.ShapeDtypeStruct((HEADS, R_TOT, 128), _f32),
                  jax.ShapeDtypeStruct((2, N_PAD, 16), _f32)],
        scratch_types=[
            pltpu.VMEM((MAC, 128), jnp.int32),
            pltpu.VMEM((MAC, 128), jnp.int32),
            pltpu.VMEM((MAC * 128, 16), _f32),
            pltpu.VMEM((MAC * 128, 16), _f32),
            pltpu.VMEM((MAC * 16, 128), _f32),
            pltpu.VMEM((MAC * 128, 16), _f32),
            pltpu.VMEM((HEADS, MAC, 128), _f32),
            pltpu.VMEM((ZR, 16), _f32),
            pltpu.VMEM_SHARED((N_PAD, 16), _f32),
            pltpu.SemaphoreType.DMA,
            pltpu.SemaphoreType.DMA,
        ],
    )
    def k(src_h, dst_h, as_h, ad_h, ae_h, ext_h, denp_h,
          si, di, ar, dr, er, xr, xtb, zb, dacc, sem1, sem2):
        cid = lax.axis_index("c")
        sid = lax.axis_index("s")

        @pl.loop(0, ZR)
        def _(i):
            zb[i, :] = jnp.zeros((16,), _f32)

        @pl.loop(0, ROWS_PER_SUB // ZR)
        def _(j):
            pltpu.sync_copy(zb, dacc.at[pl.ds(sid * ROWS_PER_SUB + j * ZR, ZR)])

        plsc.subcore_barrier()

        row0 = (cid * NSUB + sid) * RPW1
        lane = lax.iota(jnp.int32, 16)
        lane_lt4 = lane < HEADS
        scat_head = lane

        @pl.loop(0, RPW1 // MAC)
        def _(ci):
            rowb = row0 + ci * MAC
            cps = [pltpu.async_copy(src_h.at[pl.ds(rowb, MAC)], si, sem1),
                   pltpu.async_copy(dst_h.at[pl.ds(rowb, MAC)], di, sem1),
                   pltpu.async_copy(ae_h.at[pl.ds(rowb * 16, MAC * 16)],
                                    er, sem1)]
            for cp in cps:
                cp.wait()
            cps = []
            for i in range(MAC):
                cps.append(pltpu.async_copy(
                    as_h.at[si.at[i]], ar.at[pl.ds(i * 128, 128)], sem1))
                cps.append(pltpu.async_copy(
                    ad_h.at[di.at[i]], dr.at[pl.ds(i * 128, 128)], sem1))
            for cp in cps:
                cp.wait()

            @pl.loop(0, MAC * 128)
            def _(e):
                srow = ar[e, :] + dr[e, :] + er[e >> 3, pl.ds((e & 7) * 16, 16)]
                srow = jnp.maximum(srow, 0.2 * srow)
                exr = jnp.exp(srow)
                xr[e, :] = exr
                r = e >> 7
                c = e & 127
                plsc.store_scatter(
                    xtb,
                    [scat_head, jnp.full((16,), r, jnp.int32),
                     jnp.full((16,), c, jnp.int32)],
                    exr, mask=lane_lt4)

            for i in range(MAC):
                pltpu.sync_copy(xr.at[pl.ds(i * 128, 128)],
                                dacc.at[di.at[i]], add=True)
            cps = [pltpu.async_copy(xtb.at[hh],
                                    ext_h.at[hh, pl.ds(rowb, MAC)], sem2)
                   for hh in range(HEADS)]
            for cp in cps:
                cp.wait()

        plsc.subcore_barrier()
        r0 = sid * ROWS_PER_SUB
        pltpu.sync_copy(dacc.at[pl.ds(r0, ROWS_PER_SUB)],
                        denp_h.at[cid, pl.ds(r0, ROWS_PER_SUB)])

    return k(src2, dst2, asrc16, adst16, ae16)


def _sc_pass2(src2, dst2, xh8, ext3):
    @functools.partial(
        pl.kernel,
        mesh=_MESH,
        compiler_params=_SC_PARAMS,
        out_type=jax.ShapeDtypeStruct((2 * HEADS, N_PAD, 16), _f32),
        scratch_types=[
            pltpu.VMEM((MAC2, 128), jnp.int32),
            pltpu.VMEM((MAC2, 128), jnp.int32),
            pltpu.VMEM((MAC2, 128), jnp.int32),
            pltpu.VMEM((MAC2, 128), _f32),
            pltpu.VMEM((MAC2 * 128, 16), _f32),
            pltpu.VMEM((ZR, 16), _f32),
            pltpu.VMEM_SHARED((N_PAD, 16), _f32),
            pltpu.SemaphoreType.DMA,
            pltpu.SemaphoreType.DMA,
        ],
    )
    def k(src_h, dst_h, xh_h, ext_h, out_h,
          si, di, ix, exb, rows, zb, acc, sem1, sem2):
        cid = lax.axis_index("c")
        sid = lax.axis_index("s")

        @pl.loop(0, ZR)
        def _(i):
            zb[i, :] = jnp.zeros((16,), _f32)

        for hp in range(2):
            for half in range(2):
                q = cid * 4 + hp * 2 + half
                head = cid * 2 + hp

                @pl.loop(0, ROWS_PER_SUB // ZR)
                def _(j):
                    pltpu.sync_copy(
                        zb, acc.at[pl.ds(sid * ROWS_PER_SUB + j * ZR, ZR)])

                plsc.subcore_barrier()

                row0 = sid * RPS2

                @pl.loop(0, RPS2 // MAC2)
                def _(ci):
                    rowb = row0 + ci * MAC2
                    cps = [
                        pltpu.async_copy(src_h.at[pl.ds(rowb, MAC2)], si,
                                         sem1),
                        pltpu.async_copy(dst_h.at[pl.ds(rowb, MAC2)], di,
                                         sem1),
                        pltpu.async_copy(ext_h.at[head, pl.ds(rowb, MAC2)],
                                         exb, sem1),
                    ]
                    for cp in cps:
                        cp.wait()

                    @pl.loop(0, MAC2 * 8)
                    def _(g):
                        r = g >> 3
                        c = (g & 7) * 16
                        sv = si[r, pl.ds(c, 16)]
                        ix[r, pl.ds(c, 16)] = sv * (2 * HEADS) + q

                    cps = [pltpu.async_copy(xh_h.at[ix.at[i]],
                                            rows.at[pl.ds(i * 128, 128)],
                                            sem1)
                           for i in range(MAC2)]
                    for cp in cps:
                        cp.wait()

                    @pl.loop(0, MAC2 * 8)
                    def _(g):
                        r = g >> 3
                        c = (g & 7) * 16
                        exv = exb[r, pl.ds(c, 16)]
                        for e in range(16):
                            rr = r * 128 + c + e
                            rows[rr, :] = rows[rr, :] * exv[e]

                    for i in range(MAC2):
                        pltpu.sync_copy(rows.at[pl.ds(i * 128, 128)],
                                        acc.at[di.at[i]], add=True)

                plsc.subcore_barrier()
                r0 = sid * ROWS_PER_SUB
                pltpu.sync_copy(acc.at[pl.ds(r0, ROWS_PER_SUB)],
                                out_h.at[q, pl.ds(r0, ROWS_PER_SUB)])
                plsc.subcore_barrier()

    return k(src2, dst2, xh8, ext3)


def kernel(x, edge_attr, params, edge_index, batch):
    npad = E_PAD - E
    pad_src = jnp.arange(npad, dtype=jnp.int32) % N
    pad_dst = N + 104 + (jnp.arange(npad, dtype=jnp.int32) % 64)
    src2 = jnp.concatenate(
        [edge_index[0].astype(jnp.int32), pad_src]).reshape(R_TOT, 128)
    dst2 = jnp.concatenate(
        [edge_index[1].astype(jnp.int32), pad_dst]).reshape(R_TOT, 128)
    batch3 = batch.astype(jnp.int32).reshape(N // NB, 1, NB)

    p = params
    row = lambda v: v.reshape(1, -1)

    egw = p['eg_W'].reshape(LAYERS, HIDDEN // 2, HEADS, C)
    V = jnp.einsum('lkhc,lhc->lkh', egw, p['att_edge'])
    V = jnp.pad(V, ((0, 0), (0, 0), (0, 16 - HEADS)))

    eye4 = jnp.eye(HEADS, dtype=_f32)
    aM = jnp.einsum('lhc,hg->lchg', p['att_src'], eye4).reshape(
        LAYERS, HIDDEN, HEADS)
    aM = jnp.pad(aM, ((0, 0), (0, 0), (0, 16 - HEADS)))
    dM = jnp.einsum('lhc,hg->lchg', p['att_dst'], eye4).reshape(
        LAYERS, HIDDEN, HEADS)
    dM = jnp.pad(dM, ((0, 0), (0, 0), (0, 16 - HEADS)))

    R4 = jnp.kron(eye4, jnp.ones((1, C), _f32))

    h = _node_enc(x, p['ne_W'], row(p['ne_b']), row(p['ne_g']),
                  row(p['ne_beta']))
    ea_pad = jnp.concatenate(
        [edge_attr, jnp.zeros((npad, EDGE_DIM), _f32)])
    ae = _edge_enc(ea_pad, p['ee_W'], row(p['ee_b']), V)

    for l in range(LAYERS):
        xh, asrc16, adst16 = _layer_proj(h, p['gat_W'][l], aM[l], dM[l])
        ae_pk = ae[l].reshape(E_PAD // 8, 128)
        ext3, denp = _sc_pass1(src2, dst2, asrc16, adst16, ae_pk)
        xh8 = xh.reshape(2 * HEADS * N, 16)
        out8 = _sc_pass2(src2, dst2, xh8, ext3)
        h = _layer_norm_res(h, out8, denp, R4,
                            row(p['gat_b'][l]), row(p['ln_g'][l]),
                            row(p['ln_b'][l]))

    g = _pool(h, batch3)
    out = _head_mlp(g, p['f1_W'], row(p['f1_b']), p['f2_W'], row(p['f2_b']),
                    p['c1_W'], row(p['c1_b']), row(p['c2_W'][:, 0]),
                    p['c2_b'].reshape(1, 1))
    return out.reshape(NGRAPH)


# speedup vs baseline: 1.0387x; 1.0387x over previous
import dataclasses
import functools

import jax
import jax.numpy as jnp
from jax import lax
from jax.experimental import pallas as pl
from jax.experimental.pallas import tpu as pltpu
from jax.experimental.pallas import tpu_sc as plsc

N = 50000
E = 800000
NODE_DIM = 12
EDGE_DIM = 4
HIDDEN = 128
HEADS = 4
C = 32
LAYERS = 3
NGRAPH = 64

NB = 1000
EB = 2048
E_PAD = 802816
R_TOT = E_PAD // 128
MAC = 4
MAC2 = 8
NSUB = 16
NWORK = 32
N_PAD = 50176
ROWS_PER_SUB = N_PAD // NSUB
ZR = 112

_f32 = jnp.float32


def _gelu(x):
    return 0.5 * x * (1.0 + lax.erf(x * (2.0 ** -0.5)))


def _ln_rows(h, g, b):
    mu = jnp.mean(h, axis=-1, keepdims=True)
    v = jnp.mean((h - mu) ** 2, axis=-1, keepdims=True)
    return (h - mu) / jnp.sqrt(v + 1e-5) * g + b


def _node_enc(x, W, b, g, beta):
    def body(x_ref, w_ref, b_ref, g_ref, be_ref, o_ref):
        h = jnp.dot(x_ref[...], w_ref[...], preferred_element_type=_f32)
        h = h + b_ref[...]
        o_ref[...] = _gelu(_ln_rows(h, g_ref[...], be_ref[...]))

    return pl.pallas_call(
        body,
        grid=(N // NB,),
        in_specs=[
            pl.BlockSpec((NB, NODE_DIM), lambda i: (i, 0)),
            pl.BlockSpec((NODE_DIM, HIDDEN), lambda i: (0, 0)),
            pl.BlockSpec((1, HIDDEN), lambda i: (0, 0)),
            pl.BlockSpec((1, HIDDEN), lambda i: (0, 0)),
            pl.BlockSpec((1, HIDDEN), lambda i: (0, 0)),
        ],
        out_specs=pl.BlockSpec((NB, HIDDEN), lambda i: (i, 0)),
        out_shape=jax.ShapeDtypeStruct((N, HIDDEN), _f32),
    )(x, W, b, g, beta)


def _edge_enc(edge_attr, eeW, eeb, V):
    def body(ea_ref, w_ref, b_ref, v0_ref, v1_ref, v2_ref, o0, o1, o2):
        t = jnp.dot(ea_ref[...], w_ref[...], preferred_element_type=_f32)
        t = _gelu(t + b_ref[...])
        o0[...] = jnp.dot(t, v0_ref[...], preferred_element_type=_f32)
        o1[...] = jnp.dot(t, v1_ref[...], preferred_element_type=_f32)
        o2[...] = jnp.dot(t, v2_ref[...], preferred_element_type=_f32)

    os = jax.ShapeDtypeStruct((E_PAD, 16), _f32)
    return pl.pallas_call(
        body,
        grid=(E_PAD // EB,),
        in_specs=[
            pl.BlockSpec((EB, EDGE_DIM), lambda i: (i, 0)),
            pl.BlockSpec((EDGE_DIM, HIDDEN // 2), lambda i: (0, 0)),
            pl.BlockSpec((1, HIDDEN // 2), lambda i: (0, 0)),
            pl.BlockSpec((HIDDEN // 2, 16), lambda i: (0, 0)),
            pl.BlockSpec((HIDDEN // 2, 16), lambda i: (0, 0)),
            pl.BlockSpec((HIDDEN // 2, 16), lambda i: (0, 0)),
        ],
        out_specs=[pl.BlockSpec((EB, 16), lambda i: (i, 0))] * 3,
        out_shape=[os, os, os],
    )(edge_attr, eeW, eeb, V[0], V[1], V[2])


def _layer_proj(h, W, aM, dM):
    def body(h_ref, w_ref, a_ref, d_ref, xh_ref, as_ref, ad_ref):
        xh = jnp.dot(h_ref[...], w_ref[...], preferred_element_type=_f32)
        xh_ref[...] = xh
        as_ref[...] = jnp.dot(xh, a_ref[...], preferred_element_type=_f32)
        ad_ref[...] = jnp.dot(xh, d_ref[...], preferred_element_type=_f32)

    return pl.pallas_call(
        body,
        grid=(N // NB,),
        in_specs=[
            pl.BlockSpec((NB, HIDDEN), lambda i: (i, 0)),
            pl.BlockSpec((HIDDEN, HIDDEN), lambda i: (0, 0)),
            pl.BlockSpec((HIDDEN, 16), lambda i: (0, 0)),
            pl.BlockSpec((HIDDEN, 16), lambda i: (0, 0)),
        ],
        out_specs=[
            pl.BlockSpec((NB, HIDDEN), lambda i: (i, 0)),
            pl.BlockSpec((NB, 16), lambda i: (i, 0)),
            pl.BlockSpec((NB, 16), lambda i: (i, 0)),
        ],
        out_shape=[
            jax.ShapeDtypeStruct((N, HIDDEN), _f32),
            jax.ShapeDtypeStruct((N, 16), _f32),
            jax.ShapeDtypeStruct((N, 16), _f32),
        ],
    )(h, W, aM, dM)


def _layer_norm_res(h, out4, denp, R4, gb, lg, lb):
    def body(h_ref, o_ref, dp_ref, r_ref, gb_ref, g_ref, b_ref, ho_ref):
        den = dp_ref[0, :, 0:4] + dp_ref[1, :, 0:4]
        rec = 1.0 / (den + 1e-16)
        drep = jnp.dot(rec, r_ref[...], preferred_element_type=_f32)
        cat = jnp.concatenate([o_ref[q] for q in range(8)], axis=-1)
        val = h_ref[...] + cat * drep + gb_ref[...]
        ho_ref[...] = _ln_rows(val, g_ref[...], b_ref[...])

    return pl.pallas_call(
        body,
        grid=(N // NB,),
        in_specs=[
            pl.BlockSpec((NB, HIDDEN), lambda i: (i, 0)),
            pl.BlockSpec((2 * HEADS, NB, 16), lambda i: (0, i, 0)),
            pl.BlockSpec((2, NB, 16), lambda i: (0, i, 0)),
            pl.BlockSpec((HEADS, HIDDEN), lambda i: (0, 0)),
            pl.BlockSpec((1, HIDDEN), lambda i: (0, 0)),
            pl.BlockSpec((1, HIDDEN), lambda i: (0, 0)),
            pl.BlockSpec((1, HIDDEN), lambda i: (0, 0)),
        ],
        out_specs=pl.BlockSpec((NB, HIDDEN), lambda i: (i, 0)),
        out_shape=jax.ShapeDtypeStruct((N, HIDDEN), _f32),
    )(h, out4, denp, R4, gb, lg, lb)


def _pool(h, batch3):
    def body(h_ref, b_ref, o_ref, s_ref, m_ref, c_ref):
        ones = jnp.ones((NB, HIDDEN), _f32)
        i = pl.program_id(0)

        @pl.when(i == 0)
        def _():
            s_ref[...] = jnp.zeros((NGRAPH, HIDDEN), _f32)
            c_ref[...] = jnp.zeros((NGRAPH, HIDDEN), _f32)
            m_ref[...] = jnp.full((NGRAPH, HIDDEN), -jnp.inf, _f32)

        hb = h_ref[...]
        b = b_ref[0, 0, :]
        oh = (b[:, None] == lax.broadcasted_iota(
            jnp.int32, (NB, NGRAPH), 1)).astype(_f32)
        dn = (((0,), (0,)), ((), ()))
        s_ref[...] += lax.dot_general(oh, hb, dn,
                                      preferred_element_type=_f32)
        c_ref[...] += lax.dot_general(oh, ones, dn,
                                      preferred_element_type=_f32)
        mrows = [
            jnp.max(jnp.where(b[:, None] == g, hb, -jnp.inf), axis=0,
                    keepdims=True)
            for g in range(NGRAPH)
        ]
        m_ref[...] = jnp.maximum(m_ref[...], jnp.concatenate(mrows, axis=0))

        cnt = c_ref[...]
        mean = s_ref[...] / jnp.maximum(cnt, 1.0)
        mx = jnp.where(cnt > 0.0, m_ref[...], 0.0)
        o_ref[...] = jnp.concatenate([mean, mx], axis=-1)

    return pl.pallas_call(
        body,
        grid=(N // NB,),
        in_specs=[
            pl.BlockSpec((NB, HIDDEN), lambda i: (i, 0)),
            pl.BlockSpec((1, 1, NB), lambda i: (i, 0, 0)),
        ],
        out_specs=pl.BlockSpec((NGRAPH, 2 * HIDDEN), lambda i: (0, 0)),
        out_shape=jax.ShapeDtypeStruct((NGRAPH, 2 * HIDDEN), _f32),
        scratch_shapes=[
            pltpu.VMEM((NGRAPH, HIDDEN), _f32),
            pltpu.VMEM((NGRAPH, HIDDEN), _f32),
            pltpu.VMEM((NGRAPH, HIDDEN), _f32),
        ],
    )(h, batch3)


def _head_mlp(g, f1W, f1b, f2W, f2b, c1W, c1b, c2r, c2b):
    def body(g_ref, w1, b1, w2, b2, w3, b3, w4, b4, o_ref):
        x = _gelu(jnp.dot(g_ref[...], w1[...],
                          preferred_element_type=_f32) + b1[...])
        x = _gelu(jnp.dot(x, w2[...],
                          preferred_element_type=_f32) + b2[...])
        x = _gelu(jnp.dot(x, w3[...],
                          preferred_element_type=_f32) + b3[...])
        logits = jnp.sum(x * w4[...], axis=1, keepdims=True) + b4[...]
        o_ref[...] = 1.0 / (1.0 + jnp.exp(-logits))

    full = lambda s: pl.BlockSpec(s, lambda: tuple(0 for _ in s))
    return pl.pallas_call(
        body,
        in_specs=[full((NGRAPH, 2 * HIDDEN)),
                  full((2 * HIDDEN, HIDDEN)), full((1, HIDDEN)),
                  full((HIDDEN, HIDDEN // 2)), full((1, HIDDEN // 2)),
                  full((HIDDEN // 2, 64)), full((1, 64)),
                  full((1, 64)), full((1, 1))],
        out_specs=full((NGRAPH, 1)),
        out_shape=jax.ShapeDtypeStruct((NGRAPH, 1), _f32),
    )(g, f1W, f1b, f2W, f2b, c1W, c1b, c2r, c2b)


_MESH = plsc.VectorSubcoreMesh(core_axis_name="c", subcore_axis_name="s")
_SC_PARAMS = pltpu.CompilerParams()
if "needs_layout_passes" in pltpu.CompilerParams.__dataclass_fields__:
    _SC_PARAMS = dataclasses.replace(_SC_PARAMS, needs_layout_passes=False)
if "use_tc_tiling_on_sc" in pltpu.CompilerParams.__dataclass_fields__:
    _SC_PARAMS = dataclasses.replace(_SC_PARAMS, use_tc_tiling_on_sc=False)

RPW1 = R_TOT // NWORK
RPS2 = R_TOT // NSUB


def _sc_pass1(src2, dst2, asrc16, adst16, ae16):
    @functools.partial(
        pl.kernel,
        mesh=_MESH,
        compiler_params=_SC_PARAMS,
        out_type=[jax.ShapeDtypeStruct((HEADS, R_TOT, 128), _f32),
                  jax.ShapeDtypeStruct((2, N_PAD, 16), _f32)],
        scratch_types=[
            pltpu.VMEM((MAC, 128), jnp.int32),
            pltpu.VMEM((MAC, 128), jnp.int32),
            pltpu.VMEM((MAC * 128, 16), _f32),
            pltpu.VMEM((MAC * 128, 16), _f32),
            pltpu.VMEM((MAC * 16, 128), _f32),
            pltpu.VMEM((MAC * 128, 16), _f32),
            pltpu.VMEM((HEADS, MAC, 128), _f32),
            pltpu.VMEM((ZR, 16), _f32),
            pltpu.VMEM_SHARED((N_PAD, 16), _f32),
            pltpu.SemaphoreType.DMA,
            pltpu.SemaphoreType.DMA,
        ],
    )
    def k(src_h, dst_h, as_h, ad_h, ae_h, ext_h, denp_h,
          si, di, ar, dr, er, xr, xtb, zb, dacc, sem1, sem2):
        cid = lax.axis_index("c")
        sid = lax.axis_index("s")

        @pl.loop(0, ZR)
        def _(i):
            zb[i, :] = jnp.zeros((16,), _f32)

        @pl.loop(0, ROWS_PER_SUB // ZR)
        def _(j):
            pltpu.sync_copy(zb, dacc.at[pl.ds(sid * ROWS_PER_SUB + j * ZR, ZR)])

        plsc.subcore_barrier()

        row0 = (cid * NSUB + sid) * RPW1
        lane = lax.iota(jnp.int32, 16)
        lane_lt4 = lane < HEADS
        scat_head = lane

        @pl.loop(0, RPW1 // MAC)
        def _(ci):
            rowb = row0 + ci * MAC
            cps = [pltpu.async_copy(src_h.at[pl.ds(rowb, MAC)], si, sem1),
                   pltpu.async_copy(dst_h.at[pl.ds(rowb, MAC)], di, sem1),
                   pltpu.async_copy(ae_h.at[pl.ds(rowb * 16, MAC * 16)],
                                    er, sem1)]
            for cp in cps:
                cp.wait()
            cps = []
            for i in range(MAC):
                cps.append(pltpu.async_copy(
                    as_h.at[si.at[i]], ar.at[pl.ds(i * 128, 128)], sem1))
                cps.append(pltpu.async_copy(
                    ad_h.at[di.at[i]], dr.at[pl.ds(i * 128, 128)], sem1))
            for cp in cps:
                cp.wait()

            @pl.loop(0, MAC * 128)
            def _(e):
                srow = ar[e, :] + dr[e, :] + er[e >> 3, pl.ds((e & 7) * 16, 16)]
                srow = jnp.maximum(srow, 0.2 * srow)
                exr = jnp.exp(srow)
                xr[e, :] = exr
                r = e >> 7
                c = e & 127
                plsc.store_scatter(
                    xtb,
                    [scat_head, jnp.full((16,), r, jnp.int32),
                     jnp.full((16,), c, jnp.int32)],
                    exr, mask=lane_lt4)

            adds = []
            for i in range(MAC):
                adds.append(pltpu.async_copy(
                    xr.at[pl.ds(i * 128, 128)], dacc.at[di.at[i]], sem2,
                    add=True))
                if len(adds) >= 2:
                    adds.pop(0).wait()
            for cp in adds:
                cp.wait()
            cps = [pltpu.async_copy(xtb.at[hh],
                                    ext_h.at[hh, pl.ds(rowb, MAC)], sem2)
                   for hh in range(HEADS)]
            for cp in cps:
                cp.wait()

        plsc.subcore_barrier()
        r0 = sid * ROWS_PER_SUB
        pltpu.sync_copy(dacc.at[pl.ds(r0, ROWS_PER_SUB)],
                        denp_h.at[cid, pl.ds(r0, ROWS_PER_SUB)])

    return k(src2, dst2, asrc16, adst16, ae16)


def _sc_pass2(src2, dst2, xh8, ext3):
    @functools.partial(
        pl.kernel,
        mesh=_MESH,
        compiler_params=_SC_PARAMS,
        out_type=jax.ShapeDtypeStruct((2 * HEADS, N_PAD, 16), _f32),
        scratch_types=[
            pltpu.VMEM((MAC2, 128), jnp.int32),
            pltpu.VMEM((MAC2, 128), jnp.int32),
            pltpu.VMEM((MAC2, 128), jnp.int32),
            pltpu.VMEM((MAC2, 128), _f32),
            pltpu.VMEM((MAC2 * 128, 16), _f32),
            pltpu.VMEM((ZR, 16), _f32),
            pltpu.VMEM_SHARED((N_PAD, 16), _f32),
            pltpu.SemaphoreType.DMA,
            pltpu.SemaphoreType.DMA,
        ],
    )
    def k(src_h, dst_h, xh_h, ext_h, out_h,
          si, di, ix, exb, rows, zb, acc, sem1, sem2):
        cid = lax.axis_index("c")
        sid = lax.axis_index("s")

        @pl.loop(0, ZR)
        def _(i):
            zb[i, :] = jnp.zeros((16,), _f32)

        for hp in range(2):
            for half in range(2):
                q = cid * 4 + hp * 2 + half
                head = cid * 2 + hp

                @pl.loop(0, ROWS_PER_SUB // ZR)
                def _(j):
                    pltpu.sync_copy(
                        zb, acc.at[pl.ds(sid * ROWS_PER_SUB + j * ZR, ZR)])

                plsc.subcore_barrier()

                row0 = sid * RPS2

                @pl.loop(0, RPS2 // MAC2)
                def _(ci):
                    rowb = row0 + ci * MAC2
                    cps = [
                        pltpu.async_copy(src_h.at[pl.ds(rowb, MAC2)], si,
                                         sem1),
                        pltpu.async_copy(dst_h.at[pl.ds(rowb, MAC2)], di,
                                         sem1),
                        pltpu.async_copy(ext_h.at[head, pl.ds(rowb, MAC2)],
                                         exb, sem1),
                    ]
                    for cp in cps:
                        cp.wait()

                    @pl.loop(0, MAC2 * 8)
                    def _(g):
                        r = g >> 3
                        c = (g & 7) * 16
                        sv = si[r, pl.ds(c, 16)]
                        ix[r, pl.ds(c, 16)] = sv * (2 * HEADS) + q

                    cps = [pltpu.async_copy(xh_h.at[ix.at[i]],
                                            rows.at[pl.ds(i * 128, 128)],
                                            sem1)
                           for i in range(MAC2)]
                    for cp in cps:
                        cp.wait()

                    @pl.loop(0, MAC2 * 8)
                    def _(g):
                        r = g >> 3
                        c = (g & 7) * 16
                        exv = exb[r, pl.ds(c, 16)]
                        for e in range(16):
                            rr = r * 128 + c + e
                            rows[rr, :] = rows[rr, :] * exv[e]

                    adds = []
                    for i in range(MAC2):
                        adds.append(pltpu.async_copy(
                            rows.at[pl.ds(i * 128, 128)], acc.at[di.at[i]],
                            sem2, add=True))
                        if len(adds) >= 2:
                            adds.pop(0).wait()
                    for cp in adds:
                        cp.wait()

                plsc.subcore_barrier()
                r0 = sid * ROWS_PER_SUB
                pltpu.sync_copy(acc.at[pl.ds(r0, ROWS_PER_SUB)],
                                out_h.at[q, pl.ds(r0, ROWS_PER_SUB)])
                plsc.subcore_barrier()

    return k(src2, dst2, xh8, ext3)


def kernel(x, edge_attr, params, edge_index, batch):
    npad = E_PAD - E
    pad_src = jnp.arange(npad, dtype=jnp.int32) % N
    pad_dst = N + 104 + (jnp.arange(npad, dtype=jnp.int32) % 64)
    src2 = jnp.concatenate(
        [edge_index[0].astype(jnp.int32), pad_src]).reshape(R_TOT, 128)
    dst2 = jnp.concatenate(
        [edge_index[1].astype(jnp.int32), pad_dst]).reshape(R_TOT, 128)
    batch3 = batch.astype(jnp.int32).reshape(N // NB, 1, NB)

    p = params
    row = lambda v: v.reshape(1, -1)

    egw = p['eg_W'].reshape(LAYERS, HIDDEN // 2, HEADS, C)
    V = jnp.einsum('lkhc,lhc->lkh', egw, p['att_edge'])
    V = jnp.pad(V, ((0, 0), (0, 0), (0, 16 - HEADS)))

    eye4 = jnp.eye(HEADS, dtype=_f32)
    aM = jnp.einsum('lhc,hg->lchg', p['att_src'], eye4).reshape(
        LAYERS, HIDDEN, HEADS)
    aM = jnp.pad(aM, ((0, 0), (0, 0), (0, 16 - HEADS)))
    dM = jnp.einsum('lhc,hg->lchg', p['att_dst'], eye4).reshape(
        LAYERS, HIDDEN, HEADS)
    dM = jnp.pad(dM, ((0, 0), (0, 0), (0, 16 - HEADS)))

    R4 = jnp.kron(eye4, jnp.ones((1, C), _f32))

    h = _node_enc(x, p['ne_W'], row(p['ne_b']), row(p['ne_g']),
                  row(p['ne_beta']))
    ea_pad = jnp.concatenate(
        [edge_attr, jnp.zeros((npad, EDGE_DIM), _f32)])
    ae = _edge_enc(ea_pad, p['ee_W'], row(p['ee_b']), V)

    for l in range(LAYERS):
        xh, asrc16, adst16 = _layer_proj(h, p['gat_W'][l], aM[l], dM[l])
        ae_pk = ae[l].reshape(E_PAD // 8, 128)
        ext3, denp = _sc_pass1(src2, dst2, asrc16, adst16, ae_pk)
        xh8 = xh.reshape(2 * HEADS * N, 16)
        out8 = _sc_pass2(src2, dst2, xh8, ext3)
        h = _layer_norm_res(h, out8, denp, R4,
                            row(p['gat_b'][l]), row(p['ln_g'][l]),
                            row(p['ln_b'][l]))

    g = _pool(h, batch3)
    out = _head_mlp(g, p['f1_W'], row(p['f1_b']), p['f2_W'], row(p['f2_b']),
                    p['c1_W'], row(p['c1_b']), row(p['c2_W'][:, 0]),
                    p['c2_b'].reshape(1, 1))
    return out.reshape(NGRAPH)


# speedup vs baseline: 1.0860x; 1.0456x over previous
import dataclasses
import functools

import jax
import jax.numpy as jnp
from jax import lax
from jax.experimental import pallas as pl
from jax.experimental.pallas import tpu as pltpu
from jax.experimental.pallas import tpu_sc as plsc

N = 50000
E = 800000
NODE_DIM = 12
EDGE_DIM = 4
HIDDEN = 128
HEADS = 4
C = 32
LAYERS = 3
NGRAPH = 64

NB = 1000
EB = 2048
E_PAD = 802816
R_TOT = E_PAD // 128
MAC = 4
MAC2 = 14
NSUB = 16
NWORK = 32
N_PAD = 50176
ROWS_PER_SUB = N_PAD // NSUB
ZR = 112

_f32 = jnp.float32


def _gelu(x):
    return 0.5 * x * (1.0 + lax.erf(x * (2.0 ** -0.5)))


def _ln_rows(h, g, b):
    mu = jnp.mean(h, axis=-1, keepdims=True)
    v = jnp.mean((h - mu) ** 2, axis=-1, keepdims=True)
    return (h - mu) / jnp.sqrt(v + 1e-5) * g + b


def _node_enc(x, W, b, g, beta):
    def body(x_ref, w_ref, b_ref, g_ref, be_ref, o_ref):
        h = jnp.dot(x_ref[...], w_ref[...], preferred_element_type=_f32)
        h = h + b_ref[...]
        o_ref[...] = _gelu(_ln_rows(h, g_ref[...], be_ref[...]))

    return pl.pallas_call(
        body,
        grid=(N // NB,),
        in_specs=[
            pl.BlockSpec((NB, NODE_DIM), lambda i: (i, 0)),
            pl.BlockSpec((NODE_DIM, HIDDEN), lambda i: (0, 0)),
            pl.BlockSpec((1, HIDDEN), lambda i: (0, 0)),
            pl.BlockSpec((1, HIDDEN), lambda i: (0, 0)),
            pl.BlockSpec((1, HIDDEN), lambda i: (0, 0)),
        ],
        out_specs=pl.BlockSpec((NB, HIDDEN), lambda i: (i, 0)),
        out_shape=jax.ShapeDtypeStruct((N, HIDDEN), _f32),
    )(x, W, b, g, beta)


def _edge_enc(edge_attr, eeW, eeb, V):
    def body(ea_ref, w_ref, b_ref, v0_ref, v1_ref, v2_ref, o0, o1, o2):
        t = jnp.dot(ea_ref[...], w_ref[...], preferred_element_type=_f32)
        t = _gelu(t + b_ref[...])
        o0[...] = jnp.dot(t, v0_ref[...], preferred_element_type=_f32)
        o1[...] = jnp.dot(t, v1_ref[...], preferred_element_type=_f32)
        o2[...] = jnp.dot(t, v2_ref[...], preferred_element_type=_f32)

    os = jax.ShapeDtypeStruct((E_PAD, 16), _f32)
    return pl.pallas_call(
        body,
        grid=(E_PAD // EB,),
        in_specs=[
            pl.BlockSpec((EB, EDGE_DIM), lambda i: (i, 0)),
            pl.BlockSpec((EDGE_DIM, HIDDEN // 2), lambda i: (0, 0)),
            pl.BlockSpec((1, HIDDEN // 2), lambda i: (0, 0)),
            pl.BlockSpec((HIDDEN // 2, 16), lambda i: (0, 0)),
            pl.BlockSpec((HIDDEN // 2, 16), lambda i: (0, 0)),
            pl.BlockSpec((HIDDEN // 2, 16), lambda i: (0, 0)),
        ],
        out_specs=[pl.BlockSpec((EB, 16), lambda i: (i, 0))] * 3,
        out_shape=[os, os, os],
    )(edge_attr, eeW, eeb, V[0], V[1], V[2])


def _layer_proj(h, W, aM, dM):
    def body(h_ref, w_ref, a_ref, d_ref, xh_ref, as_ref, ad_ref):
        xh = jnp.dot(h_ref[...], w_ref[...], preferred_element_type=_f32)
        xh_ref[...] = xh
        as_ref[...] = jnp.dot(xh, a_ref[...], preferred_element_type=_f32)
        ad_ref[...] = jnp.dot(xh, d_ref[...], preferred_element_type=_f32)

    return pl.pallas_call(
        body,
        grid=(N // NB,),
        in_specs=[
            pl.BlockSpec((NB, HIDDEN), lambda i: (i, 0)),
            pl.BlockSpec((HIDDEN, HIDDEN), lambda i: (0, 0)),
            pl.BlockSpec((HIDDEN, 16), lambda i: (0, 0)),
            pl.BlockSpec((HIDDEN, 16), lambda i: (0, 0)),
        ],
        out_specs=[
            pl.BlockSpec((NB, HIDDEN), lambda i: (i, 0)),
            pl.BlockSpec((NB, 16), lambda i: (i, 0)),
            pl.BlockSpec((NB, 16), lambda i: (i, 0)),
        ],
        out_shape=[
            jax.ShapeDtypeStruct((N, HIDDEN), _f32),
            jax.ShapeDtypeStruct((N, 16), _f32),
            jax.ShapeDtypeStruct((N, 16), _f32),
        ],
    )(h, W, aM, dM)


def _layer_norm_res(h, out4, denp, R4, gb, lg, lb):
    def body(h_ref, o_ref, dp_ref, r_ref, gb_ref, g_ref, b_ref, ho_ref):
        den = dp_ref[0, :, 0:4] + dp_ref[1, :, 0:4]
        rec = 1.0 / (den + 1e-16)
        drep = jnp.dot(rec, r_ref[...], preferred_element_type=_f32)
        cat = jnp.concatenate([o_ref[q] for q in range(8)], axis=-1)
        val = h_ref[...] + cat * drep + gb_ref[...]
        ho_ref[...] = _ln_rows(val, g_ref[...], b_ref[...])

    return pl.pallas_call(
        body,
        grid=(N // NB,),
        in_specs=[
            pl.BlockSpec((NB, HIDDEN), lambda i: (i, 0)),
            pl.BlockSpec((2 * HEADS, NB, 16), lambda i: (0, i, 0)),
            pl.BlockSpec((2, NB, 16), lambda i: (0, i, 0)),
            pl.BlockSpec((HEADS, HIDDEN), lambda i: (0, 0)),
            pl.BlockSpec((1, HIDDEN), lambda i: (0, 0)),
            pl.BlockSpec((1, HIDDEN), lambda i: (0, 0)),
            pl.BlockSpec((1, HIDDEN), lambda i: (0, 0)),
        ],
        out_specs=pl.BlockSpec((NB, HIDDEN), lambda i: (i, 0)),
        out_shape=jax.ShapeDtypeStruct((N, HIDDEN), _f32),
    )(h, out4, denp, R4, gb, lg, lb)


def _pool(h, batch3):
    def body(h_ref, b_ref, o_ref, s_ref, m_ref, c_ref):
        ones = jnp.ones((NB, HIDDEN), _f32)
        i = pl.program_id(0)

        @pl.when(i == 0)
        def _():
            s_ref[...] = jnp.zeros((NGRAPH, HIDDEN), _f32)
            c_ref[...] = jnp.zeros((NGRAPH, HIDDEN), _f32)
            m_ref[...] = jnp.full((NGRAPH, HIDDEN), -jnp.inf, _f32)

        hb = h_ref[...]
        b = b_ref[0, 0, :]
        oh = (b[:, None] == lax.broadcasted_iota(
            jnp.int32, (NB, NGRAPH), 1)).astype(_f32)
        dn = (((0,), (0,)), ((), ()))
        s_ref[...] += lax.dot_general(oh, hb, dn,
                                      preferred_element_type=_f32)
        c_ref[...] += lax.dot_general(oh, ones, dn,
                                      preferred_element_type=_f32)
        mrows = [
            jnp.max(jnp.where(b[:, None] == g, hb, -jnp.inf), axis=0,
                    keepdims=True)
            for g in range(NGRAPH)
        ]
        m_ref[...] = jnp.maximum(m_ref[...], jnp.concatenate(mrows, axis=0))

        cnt = c_ref[...]
        mean = s_ref[...] / jnp.maximum(cnt, 1.0)
        mx = jnp.where(cnt > 0.0, m_ref[...], 0.0)
        o_ref[...] = jnp.concatenate([mean, mx], axis=-1)

    return pl.pallas_call(
        body,
        grid=(N // NB,),
        in_specs=[
            pl.BlockSpec((NB, HIDDEN), lambda i: (i, 0)),
            pl.BlockSpec((1, 1, NB), lambda i: (i, 0, 0)),
        ],
        out_specs=pl.BlockSpec((NGRAPH, 2 * HIDDEN), lambda i: (0, 0)),
        out_shape=jax.ShapeDtypeStruct((NGRAPH, 2 * HIDDEN), _f32),
        scratch_shapes=[
            pltpu.VMEM((NGRAPH, HIDDEN), _f32),
            pltpu.VMEM((NGRAPH, HIDDEN), _f32),
            pltpu.VMEM((NGRAPH, HIDDEN), _f32),
        ],
    )(h, batch3)


def _head_mlp(g, f1W, f1b, f2W, f2b, c1W, c1b, c2r, c2b):
    def body(g_ref, w1, b1, w2, b2, w3, b3, w4, b4, o_ref):
        x = _gelu(jnp.dot(g_ref[...], w1[...],
                          preferred_element_type=_f32) + b1[...])
        x = _gelu(jnp.dot(x, w2[...],
                          preferred_element_type=_f32) + b2[...])
        x = _gelu(jnp.dot(x, w3[...],
                          preferred_element_type=_f32) + b3[...])
        logits = jnp.sum(x * w4[...], axis=1, keepdims=True) + b4[...]
        o_ref[...] = 1.0 / (1.0 + jnp.exp(-logits))

    full = lambda s: pl.BlockSpec(s, lambda: tuple(0 for _ in s))
    return pl.pallas_call(
        body,
        in_specs=[full((NGRAPH, 2 * HIDDEN)),
                  full((2 * HIDDEN, HIDDEN)), full((1, HIDDEN)),
                  full((HIDDEN, HIDDEN // 2)), full((1, HIDDEN // 2)),
                  full((HIDDEN // 2, 64)), full((1, 64)),
                  full((1, 64)), full((1, 1))],
        out_specs=full((NGRAPH, 1)),
        out_shape=jax.ShapeDtypeStruct((NGRAPH, 1), _f32),
    )(g, f1W, f1b, f2W, f2b, c1W, c1b, c2r, c2b)


_MESH = plsc.VectorSubcoreMesh(core_axis_name="c", subcore_axis_name="s")
_SC_PARAMS = pltpu.CompilerParams()
if "needs_layout_passes" in pltpu.CompilerParams.__dataclass_fields__:
    _SC_PARAMS = dataclasses.replace(_SC_PARAMS, needs_layout_passes=False)
if "use_tc_tiling_on_sc" in pltpu.CompilerParams.__dataclass_fields__:
    _SC_PARAMS = dataclasses.replace(_SC_PARAMS, use_tc_tiling_on_sc=False)

RPW1 = R_TOT // NWORK
RPS2 = R_TOT // NSUB


def _sc_pass1(src2, dst2, asrc16, adst16, ae16):
    @functools.partial(
        pl.kernel,
        mesh=_MESH,
        compiler_params=_SC_PARAMS,
        out_type=[jax.ShapeDtypeStruct((HEADS, R_TOT, 128), _f32),
                  jax.ShapeDtypeStruct((2, N_PAD, 16), _f32)],
        scratch_types=[
            pltpu.VMEM((MAC, 128), jnp.int32),
            pltpu.VMEM((MAC, 128), jnp.int32),
            pltpu.VMEM((MAC * 128, 16), _f32),
            pltpu.VMEM((MAC * 128, 16), _f32),
            pltpu.VMEM((MAC * 16, 128), _f32),
            pltpu.VMEM((MAC * 128, 16), _f32),
            pltpu.VMEM((HEADS, MAC, 128), _f32),
            pltpu.VMEM((ZR, 16), _f32),
            pltpu.VMEM_SHARED((N_PAD, 16), _f32),
            pltpu.SemaphoreType.DMA,
            pltpu.SemaphoreType.DMA,
        ],
    )
    def k(src_h, dst_h, as_h, ad_h, ae_h, ext_h, denp_h,
          si, di, ar, dr, er, xr, xtb, zb, dacc, sem1, sem2):
        cid = lax.axis_index("c")
        sid = lax.axis_index("s")

        @pl.loop(0, ZR)
        def _(i):
            zb[i, :] = jnp.zeros((16,), _f32)

        @pl.loop(0, ROWS_PER_SUB // ZR)
        def _(j):
            pltpu.sync_copy(zb, dacc.at[pl.ds(sid * ROWS_PER_SUB + j * ZR, ZR)])

        plsc.subcore_barrier()

        row0 = (cid * NSUB + sid) * RPW1
        lane = lax.iota(jnp.int32, 16)
        lane_lt4 = lane < HEADS
        scat_head = lane

        @pl.loop(0, RPW1 // MAC)
        def _(ci):
            rowb = row0 + ci * MAC
            cps = [pltpu.async_copy(src_h.at[pl.ds(rowb, MAC)], si, sem1),
                   pltpu.async_copy(dst_h.at[pl.ds(rowb, MAC)], di, sem1),
                   pltpu.async_copy(ae_h.at[pl.ds(rowb * 16, MAC * 16)],
                                    er, sem1)]
            for cp in cps:
                cp.wait()
            cps = []
            for i in range(MAC):
                cps.append(pltpu.async_copy(
                    as_h.at[si.at[i]], ar.at[pl.ds(i * 128, 128)], sem1))
                cps.append(pltpu.async_copy(
                    ad_h.at[di.at[i]], dr.at[pl.ds(i * 128, 128)], sem1))
            for cp in cps:
                cp.wait()

            @pl.loop(0, MAC * 128)
            def _(e):
                srow = ar[e, :] + dr[e, :] + er[e >> 3, pl.ds((e & 7) * 16, 16)]
                srow = jnp.maximum(srow, 0.2 * srow)
                exr = jnp.exp(srow)
                xr[e, :] = exr
                r = e >> 7
                c = e & 127
                plsc.store_scatter(
                    xtb,
                    [scat_head, jnp.full((16,), r, jnp.int32),
                     jnp.full((16,), c, jnp.int32)],
                    exr, mask=lane_lt4)

            adds = []
            for i in range(MAC):
                adds.append(pltpu.async_copy(
                    xr.at[pl.ds(i * 128, 128)], dacc.at[di.at[i]], sem2,
                    add=True))
                if len(adds) >= 4:
                    adds.pop(0).wait()
            for cp in adds:
                cp.wait()
            cps = [pltpu.async_copy(xtb.at[hh],
                                    ext_h.at[hh, pl.ds(rowb, MAC)], sem2)
                   for hh in range(HEADS)]
            for cp in cps:
                cp.wait()

        plsc.subcore_barrier()
        r0 = sid * ROWS_PER_SUB
        pltpu.sync_copy(dacc.at[pl.ds(r0, ROWS_PER_SUB)],
                        denp_h.at[cid, pl.ds(r0, ROWS_PER_SUB)])

    return k(src2, dst2, asrc16, adst16, ae16)


def _sc_pass2(src2, dst2, xh8, ext3):
    @functools.partial(
        pl.kernel,
        mesh=_MESH,
        compiler_params=_SC_PARAMS,
        out_type=jax.ShapeDtypeStruct((2 * HEADS, N_PAD, 16), _f32),
        scratch_types=[
            pltpu.VMEM((MAC2, 128), jnp.int32),
            pltpu.VMEM((MAC2, 128), jnp.int32),
            pltpu.VMEM((MAC2, 128), jnp.int32),
            pltpu.VMEM((MAC2, 128), _f32),
            pltpu.VMEM((MAC2 * 128, 16), _f32),
            pltpu.VMEM((ZR, 16), _f32),
            pltpu.VMEM_SHARED((N_PAD, 16), _f32),
            pltpu.SemaphoreType.DMA,
            pltpu.SemaphoreType.DMA,
        ],
    )
    def k(src_h, dst_h, xh_h, ext_h, out_h,
          si, di, ix, exb, rows, zb, acc, sem1, sem2):
        cid = lax.axis_index("c")
        sid = lax.axis_index("s")

        @pl.loop(0, ZR)
        def _(i):
            zb[i, :] = jnp.zeros((16,), _f32)

        for hp in range(2):
            for half in range(2):
                q = cid * 4 + hp * 2 + half
                head = cid * 2 + hp

                @pl.loop(0, ROWS_PER_SUB // ZR)
                def _(j):
                    pltpu.sync_copy(
                        zb, acc.at[pl.ds(sid * ROWS_PER_SUB + j * ZR, ZR)])

                plsc.subcore_barrier()

                row0 = sid * RPS2

                @pl.loop(0, RPS2 // MAC2)
                def _(ci):
                    rowb = row0 + ci * MAC2
                    cps = [
                        pltpu.async_copy(src_h.at[pl.ds(rowb, MAC2)], si,
                                         sem1),
                        pltpu.async_copy(dst_h.at[pl.ds(rowb, MAC2)], di,
                                         sem1),
                        pltpu.async_copy(ext_h.at[head, pl.ds(rowb, MAC2)],
                                         exb, sem1),
                    ]
                    for cp in cps:
                        cp.wait()

                    @pl.loop(0, MAC2 * 8)
                    def _(g):
                        r = g >> 3
                        c = (g & 7) * 16
                        sv = si[r, pl.ds(c, 16)]
                        ix[r, pl.ds(c, 16)] = sv * (2 * HEADS) + q

                    cps = [pltpu.async_copy(xh_h.at[ix.at[i]],
                                            rows.at[pl.ds(i * 128, 128)],
                                            sem1)
                           for i in range(MAC2)]
                    for cp in cps:
                        cp.wait()

                    @pl.loop(0, MAC2 * 8)
                    def _(g):
                        r = g >> 3
                        c = (g & 7) * 16
                        exv = exb[r, pl.ds(c, 16)]
                        for e in range(16):
                            rr = r * 128 + c + e
                            rows[rr, :] = rows[rr, :] * exv[e]

                    adds = []
                    for i in range(MAC2):
                        adds.append(pltpu.async_copy(
                            rows.at[pl.ds(i * 128, 128)], acc.at[di.at[i]],
                            sem2, add=True))
                        if len(adds) >= 4:
                            adds.pop(0).wait()
                    for cp in adds:
                        cp.wait()

                plsc.subcore_barrier()
                r0 = sid * ROWS_PER_SUB
                pltpu.sync_copy(acc.at[pl.ds(r0, ROWS_PER_SUB)],
                                out_h.at[q, pl.ds(r0, ROWS_PER_SUB)])
                plsc.subcore_barrier()

    return k(src2, dst2, xh8, ext3)


def kernel(x, edge_attr, params, edge_index, batch):
    npad = E_PAD - E
    pad_src = jnp.arange(npad, dtype=jnp.int32) % N
    pad_dst = N + 104 + (jnp.arange(npad, dtype=jnp.int32) % 64)
    src2 = jnp.concatenate(
        [edge_index[0].astype(jnp.int32), pad_src]).reshape(R_TOT, 128)
    dst2 = jnp.concatenate(
        [edge_index[1].astype(jnp.int32), pad_dst]).reshape(R_TOT, 128)
    batch3 = batch.astype(jnp.int32).reshape(N // NB, 1, NB)

    p = params
    row = lambda v: v.reshape(1, -1)

    egw = p['eg_W'].reshape(LAYERS, HIDDEN // 2, HEADS, C)
    V = jnp.einsum('lkhc,lhc->lkh', egw, p['att_edge'])
    V = jnp.pad(V, ((0, 0), (0, 0), (0, 16 - HEADS)))

    eye4 = jnp.eye(HEADS, dtype=_f32)
    aM = jnp.einsum('lhc,hg->lchg', p['att_src'], eye4).reshape(
        LAYERS, HIDDEN, HEADS)
    aM = jnp.pad(aM, ((0, 0), (0, 0), (0, 16 - HEADS)))
    dM = jnp.einsum('lhc,hg->lchg', p['att_dst'], eye4).reshape(
        LAYERS, HIDDEN, HEADS)
    dM = jnp.pad(dM, ((0, 0), (0, 0), (0, 16 - HEADS)))

    R4 = jnp.kron(eye4, jnp.ones((1, C), _f32))

    h = _node_enc(x, p['ne_W'], row(p['ne_b']), row(p['ne_g']),
                  row(p['ne_beta']))
    ea_pad = jnp.concatenate(
        [edge_attr, jnp.zeros((npad, EDGE_DIM), _f32)])
    ae = _edge_enc(ea_pad, p['ee_W'], row(p['ee_b']), V)

    for l in range(LAYERS):
        xh, asrc16, adst16 = _layer_proj(h, p['gat_W'][l], aM[l], dM[l])
        ae_pk = ae[l].reshape(E_PAD // 8, 128)
        ext3, denp = _sc_pass1(src2, dst2, asrc16, adst16, ae_pk)
        xh8 = xh.reshape(2 * HEADS * N, 16)
        out8 = _sc_pass2(src2, dst2, xh8, ext3)
        h = _layer_norm_res(h, out8, denp, R4,
                            row(p['gat_b'][l]), row(p['ln_g'][l]),
                            row(p['ln_b'][l]))

    g = _pool(h, batch3)
    out = _head_mlp(g, p['f1_W'], row(p['f1_b']), p['f2_W'], row(p['f2_b']),
                    p['c1_W'], row(p['c1_b']), row(p['c2_W'][:, 0]),
                    p['c2_b'].reshape(1, 1))
    return out.reshape(NGRAPH)


# speedup vs baseline: 1.2166x; 1.1203x over previous
import dataclasses
import functools

import jax
import jax.numpy as jnp
from jax import lax
from jax.experimental import pallas as pl
from jax.experimental.pallas import tpu as pltpu
from jax.experimental.pallas import tpu_sc as plsc

N = 50000
E = 800000
NODE_DIM = 12
EDGE_DIM = 4
HIDDEN = 128
HEADS = 4
C = 32
LAYERS = 3
NGRAPH = 64

NB = 1000
EB = 2000
E_PAD = 802816
R_TOT = E_PAD // 128
MAC = 7
MAC2 = 14
NSUB = 16
NWORK = 32
N_PAD = 50176
ROWS_PER_SUB = N_PAD // NSUB
ZR = 112

_f32 = jnp.float32


def _gelu(x):
    return 0.5 * x * (1.0 + lax.erf(x * (2.0 ** -0.5)))


def _ln_rows(h, g, b):
    mu = jnp.mean(h, axis=-1, keepdims=True)
    v = jnp.mean((h - mu) ** 2, axis=-1, keepdims=True)
    return (h - mu) / jnp.sqrt(v + 1e-5) * g + b


def _node_enc(x, W, b, g, beta):
    def body(x_ref, w_ref, b_ref, g_ref, be_ref, o_ref):
        h = jnp.dot(x_ref[...], w_ref[...], preferred_element_type=_f32)
        h = h + b_ref[...]
        o_ref[...] = _gelu(_ln_rows(h, g_ref[...], be_ref[...]))

    return pl.pallas_call(
        body,
        grid=(N // NB,),
        in_specs=[
            pl.BlockSpec((NB, NODE_DIM), lambda i: (i, 0)),
            pl.BlockSpec((NODE_DIM, HIDDEN), lambda i: (0, 0)),
            pl.BlockSpec((1, HIDDEN), lambda i: (0, 0)),
            pl.BlockSpec((1, HIDDEN), lambda i: (0, 0)),
            pl.BlockSpec((1, HIDDEN), lambda i: (0, 0)),
        ],
        out_specs=pl.BlockSpec((NB, HIDDEN), lambda i: (i, 0)),
        out_shape=jax.ShapeDtypeStruct((N, HIDDEN), _f32),
    )(x, W, b, g, beta)


def _edge_enc(edge_attr, eeW, eeb, V):
    def body(ea_ref, w_ref, b_ref, v0_ref, v1_ref, v2_ref, o0, o1, o2):
        t = jnp.dot(ea_ref[...], w_ref[...], preferred_element_type=_f32)
        t = _gelu(t + b_ref[...])
        o0[...] = jnp.dot(t, v0_ref[...], preferred_element_type=_f32)
        o1[...] = jnp.dot(t, v1_ref[...], preferred_element_type=_f32)
        o2[...] = jnp.dot(t, v2_ref[...], preferred_element_type=_f32)

    os = jax.ShapeDtypeStruct((E_PAD, 16), _f32)
    return pl.pallas_call(
        body,
        grid=(E // EB,),
        in_specs=[
            pl.BlockSpec((EB, EDGE_DIM), lambda i: (i, 0)),
            pl.BlockSpec((EDGE_DIM, HIDDEN // 2), lambda i: (0, 0)),
            pl.BlockSpec((1, HIDDEN // 2), lambda i: (0, 0)),
            pl.BlockSpec((HIDDEN // 2, 16), lambda i: (0, 0)),
            pl.BlockSpec((HIDDEN // 2, 16), lambda i: (0, 0)),
            pl.BlockSpec((HIDDEN // 2, 16), lambda i: (0, 0)),
        ],
        out_specs=[pl.BlockSpec((EB, 16), lambda i: (i, 0))] * 3,
        out_shape=[os, os, os],
    )(edge_attr, eeW, eeb, V[0], V[1], V[2])


def _layer_proj(h, W, aM, dM):
    def body(h_ref, w_ref, a_ref, d_ref, xh_ref, as_ref, ad_ref):
        xh = jnp.dot(h_ref[...], w_ref[...], preferred_element_type=_f32)
        xh_ref[...] = xh
        as_ref[...] = jnp.dot(xh, a_ref[...], preferred_element_type=_f32)
        ad_ref[...] = jnp.dot(xh, d_ref[...], preferred_element_type=_f32)

    return pl.pallas_call(
        body,
        grid=(N // NB,),
        in_specs=[
            pl.BlockSpec((NB, HIDDEN), lambda i: (i, 0)),
            pl.BlockSpec((HIDDEN, HIDDEN), lambda i: (0, 0)),
            pl.BlockSpec((HIDDEN, 16), lambda i: (0, 0)),
            pl.BlockSpec((HIDDEN, 16), lambda i: (0, 0)),
        ],
        out_specs=[
            pl.BlockSpec((NB, HIDDEN), lambda i: (i, 0)),
            pl.BlockSpec((NB, 16), lambda i: (i, 0)),
            pl.BlockSpec((NB, 16), lambda i: (i, 0)),
        ],
        out_shape=[
            jax.ShapeDtypeStruct((N, HIDDEN), _f32),
            jax.ShapeDtypeStruct((N, 16), _f32),
            jax.ShapeDtypeStruct((N, 16), _f32),
        ],
    )(h, W, aM, dM)


def _layer_norm_res(h, out4, denp, R4, gb, lg, lb):
    def body(h_ref, o_ref, dp_ref, r_ref, gb_ref, g_ref, b_ref, ho_ref):
        den = dp_ref[0, :, 0:4] + dp_ref[1, :, 0:4]
        rec = 1.0 / (den + 1e-16)
        drep = jnp.dot(rec, r_ref[...], preferred_element_type=_f32)
        cat = jnp.concatenate([o_ref[q] for q in range(8)], axis=-1)
        val = h_ref[...] + cat * drep + gb_ref[...]
        ho_ref[...] = _ln_rows(val, g_ref[...], b_ref[...])

    return pl.pallas_call(
        body,
        grid=(N // NB,),
        in_specs=[
            pl.BlockSpec((NB, HIDDEN), lambda i: (i, 0)),
            pl.BlockSpec((2 * HEADS, NB, 16), lambda i: (0, i, 0)),
            pl.BlockSpec((2, NB, 16), lambda i: (0, i, 0)),
            pl.BlockSpec((HEADS, HIDDEN), lambda i: (0, 0)),
            pl.BlockSpec((1, HIDDEN), lambda i: (0, 0)),
            pl.BlockSpec((1, HIDDEN), lambda i: (0, 0)),
            pl.BlockSpec((1, HIDDEN), lambda i: (0, 0)),
        ],
        out_specs=pl.BlockSpec((NB, HIDDEN), lambda i: (i, 0)),
        out_shape=jax.ShapeDtypeStruct((N, HIDDEN), _f32),
    )(h, out4, denp, R4, gb, lg, lb)


def _pool(h, batch3):
    def body(h_ref, b_ref, o_ref, s_ref, m_ref, c_ref):
        ones = jnp.ones((NB, HIDDEN), _f32)
        i = pl.program_id(0)

        @pl.when(i == 0)
        def _():
            s_ref[...] = jnp.zeros((NGRAPH, HIDDEN), _f32)
            c_ref[...] = jnp.zeros((NGRAPH, HIDDEN), _f32)
            m_ref[...] = jnp.full((NGRAPH, HIDDEN), -jnp.inf, _f32)

        hb = h_ref[...]
        b = b_ref[0, 0, :]
        oh = (b[:, None] == lax.broadcasted_iota(
            jnp.int32, (NB, NGRAPH), 1)).astype(_f32)
        dn = (((0,), (0,)), ((), ()))
        s_ref[...] += lax.dot_general(oh, hb, dn,
                                      preferred_element_type=_f32)
        c_ref[...] += lax.dot_general(oh, ones, dn,
                                      preferred_element_type=_f32)
        mrows = [
            jnp.max(jnp.where(b[:, None] == g, hb, -jnp.inf), axis=0,
                    keepdims=True)
            for g in range(NGRAPH)
        ]
        m_ref[...] = jnp.maximum(m_ref[...], jnp.concatenate(mrows, axis=0))

        cnt = c_ref[...]
        mean = s_ref[...] / jnp.maximum(cnt, 1.0)
        mx = jnp.where(cnt > 0.0, m_ref[...], 0.0)
        o_ref[...] = jnp.concatenate([mean, mx], axis=-1)

    return pl.pallas_call(
        body,
        grid=(N // NB,),
        in_specs=[
            pl.BlockSpec((NB, HIDDEN), lambda i: (i, 0)),
            pl.BlockSpec((1, 1, NB), lambda i: (i, 0, 0)),
        ],
        out_specs=pl.BlockSpec((NGRAPH, 2 * HIDDEN), lambda i: (0, 0)),
        out_shape=jax.ShapeDtypeStruct((NGRAPH, 2 * HIDDEN), _f32),
        scratch_shapes=[
            pltpu.VMEM((NGRAPH, HIDDEN), _f32),
            pltpu.VMEM((NGRAPH, HIDDEN), _f32),
            pltpu.VMEM((NGRAPH, HIDDEN), _f32),
        ],
    )(h, batch3)


def _head_mlp(g, f1W, f1b, f2W, f2b, c1W, c1b, c2r, c2b):
    def body(g_ref, w1, b1, w2, b2, w3, b3, w4, b4, o_ref):
        x = _gelu(jnp.dot(g_ref[...], w1[...],
                          preferred_element_type=_f32) + b1[...])
        x = _gelu(jnp.dot(x, w2[...],
                          preferred_element_type=_f32) + b2[...])
        x = _gelu(jnp.dot(x, w3[...],
                          preferred_element_type=_f32) + b3[...])
        logits = jnp.sum(x * w4[...], axis=1, keepdims=True) + b4[...]
        o_ref[...] = 1.0 / (1.0 + jnp.exp(-logits))

    full = lambda s: pl.BlockSpec(s, lambda: tuple(0 for _ in s))
    return pl.pallas_call(
        body,
        in_specs=[full((NGRAPH, 2 * HIDDEN)),
                  full((2 * HIDDEN, HIDDEN)), full((1, HIDDEN)),
                  full((HIDDEN, HIDDEN // 2)), full((1, HIDDEN // 2)),
                  full((HIDDEN // 2, 64)), full((1, 64)),
                  full((1, 64)), full((1, 1))],
        out_specs=full((NGRAPH, 1)),
        out_shape=jax.ShapeDtypeStruct((NGRAPH, 1), _f32),
    )(g, f1W, f1b, f2W, f2b, c1W, c1b, c2r, c2b)


_MESH = plsc.VectorSubcoreMesh(core_axis_name="c", subcore_axis_name="s")
_SC_PARAMS = pltpu.CompilerParams()
if "needs_layout_passes" in pltpu.CompilerParams.__dataclass_fields__:
    _SC_PARAMS = dataclasses.replace(_SC_PARAMS, needs_layout_passes=False)
if "use_tc_tiling_on_sc" in pltpu.CompilerParams.__dataclass_fields__:
    _SC_PARAMS = dataclasses.replace(_SC_PARAMS, use_tc_tiling_on_sc=False)

RPW1 = R_TOT // NWORK
RPS2 = R_TOT // NSUB


def _sc_pass1(src2, dst2, asrc16, adst16, ae16):
    @functools.partial(
        pl.kernel,
        mesh=_MESH,
        compiler_params=_SC_PARAMS,
        out_type=[jax.ShapeDtypeStruct((HEADS * E_PAD,), _f32),
                  jax.ShapeDtypeStruct((2, N_PAD, 16), _f32)],
        scratch_types=[
            pltpu.VMEM((MAC, 128), jnp.int32),
            pltpu.VMEM((MAC, 128), jnp.int32),
            pltpu.VMEM((MAC * 128, 16), _f32),
            pltpu.VMEM((MAC * 128, 16), _f32),
            pltpu.VMEM((MAC * 16, 128), _f32),
            pltpu.VMEM((MAC * 128, 16), _f32),
            pltpu.VMEM((HEADS * MAC * 128,), _f32),
            pltpu.VMEM((ZR, 16), _f32),
            pltpu.VMEM_SHARED((N_PAD, 16), _f32),
            pltpu.SemaphoreType.DMA,
            pltpu.SemaphoreType.DMA,
        ],
    )
    def k(src_h, dst_h, as_h, ad_h, ae_h, ext_h, denp_h,
          si, di, ar, dr, er, xr, xtb, zb, dacc, sem1, sem2):
        cid = lax.axis_index("c")
        sid = lax.axis_index("s")

        @pl.loop(0, ZR)
        def _(i):
            zb[i, :] = jnp.zeros((16,), _f32)

        @pl.loop(0, ROWS_PER_SUB // ZR)
        def _(j):
            pltpu.sync_copy(zb, dacc.at[pl.ds(sid * ROWS_PER_SUB + j * ZR, ZR)])

        plsc.subcore_barrier()

        row0 = (cid * NSUB + sid) * RPW1
        lane = lax.iota(jnp.int32, 16)
        lane_lt4 = lane < HEADS
        scat_pat = lane * (MAC * 128)

        @pl.loop(0, RPW1 // MAC)
        def _(ci):
            rowb = row0 + ci * MAC
            cps = [pltpu.async_copy(src_h.at[pl.ds(rowb, MAC)], si, sem1),
                   pltpu.async_copy(dst_h.at[pl.ds(rowb, MAC)], di, sem1),
                   pltpu.async_copy(ae_h.at[pl.ds(rowb * 16, MAC * 16)],
                                    er, sem1)]
            for cp in cps:
                cp.wait()
            cps = []
            for i in range(MAC):
                cps.append(pltpu.async_copy(
                    as_h.at[si.at[i]], ar.at[pl.ds(i * 128, 128)], sem1))
                cps.append(pltpu.async_copy(
                    ad_h.at[di.at[i]], dr.at[pl.ds(i * 128, 128)], sem1))
            for cp in cps:
                cp.wait()

            @pl.loop(0, MAC * 128)
            def _(e):
                srow = ar[e, :] + dr[e, :] + er[e >> 3, pl.ds((e & 7) * 16, 16)]
                srow = jnp.maximum(srow, 0.2 * srow)
                exr = jnp.exp(srow)
                xr[e, :] = exr
                plsc.store_scatter(xtb, [scat_pat + e], exr, mask=lane_lt4)

            adds = []
            for i in range(MAC):
                adds.append(pltpu.async_copy(
                    xr.at[pl.ds(i * 128, 128)], dacc.at[di.at[i]], sem2,
                    add=True))
                if len(adds) >= 4:
                    adds.pop(0).wait()
            for cp in adds:
                cp.wait()
            cps = [pltpu.async_copy(
                xtb.at[pl.ds(hh * MAC * 128, MAC * 128)],
                ext_h.at[pl.ds(hh * E_PAD + rowb * 128, MAC * 128)], sem2)
                   for hh in range(HEADS)]
            for cp in cps:
                cp.wait()

        plsc.subcore_barrier()
        r0 = sid * ROWS_PER_SUB
        pltpu.sync_copy(dacc.at[pl.ds(r0, ROWS_PER_SUB)],
                        denp_h.at[cid, pl.ds(r0, ROWS_PER_SUB)])

    return k(src2, dst2, asrc16, adst16, ae16)


def _sc_pass2(src2, dst2, xh8, ext3):
    @functools.partial(
        pl.kernel,
        mesh=_MESH,
        compiler_params=_SC_PARAMS,
        out_type=jax.ShapeDtypeStruct((2 * HEADS, N_PAD, 16), _f32),
        scratch_types=[
            pltpu.VMEM((MAC2, 128), jnp.int32),
            pltpu.VMEM((MAC2, 128), jnp.int32),
            pltpu.VMEM((MAC2, 128), jnp.int32),
            pltpu.VMEM((MAC2 * 128,), _f32),
            pltpu.VMEM((MAC2 * 128, 16), _f32),
            pltpu.VMEM((ZR, 16), _f32),
            pltpu.VMEM_SHARED((N_PAD, 16), _f32),
            pltpu.SemaphoreType.DMA,
            pltpu.SemaphoreType.DMA,
        ],
    )
    def k(src_h, dst_h, xh_h, ext_h, out_h,
          si, di, ix, exb, rows, zb, acc, sem1, sem2):
        cid = lax.axis_index("c")
        sid = lax.axis_index("s")

        @pl.loop(0, ZR)
        def _(i):
            zb[i, :] = jnp.zeros((16,), _f32)

        for hp in range(2):
            for half in range(2):
                q = cid * 4 + hp * 2 + half
                head = cid * 2 + hp

                @pl.loop(0, ROWS_PER_SUB // ZR)
                def _(j):
                    pltpu.sync_copy(
                        zb, acc.at[pl.ds(sid * ROWS_PER_SUB + j * ZR, ZR)])

                plsc.subcore_barrier()

                row0 = sid * RPS2

                @pl.loop(0, RPS2 // MAC2)
                def _(ci):
                    rowb = row0 + ci * MAC2
                    cps = [
                        pltpu.async_copy(src_h.at[pl.ds(rowb, MAC2)], si,
                                         sem1),
                        pltpu.async_copy(dst_h.at[pl.ds(rowb, MAC2)], di,
                                         sem1),
                        pltpu.async_copy(
                            ext_h.at[pl.ds(head * E_PAD + rowb * 128,
                                           MAC2 * 128)], exb, sem1),
                    ]
                    for cp in cps:
                        cp.wait()

                    @pl.loop(0, MAC2 * 8)
                    def _(g):
                        r = g >> 3
                        c = (g & 7) * 16
                        sv = si[r, pl.ds(c, 16)]
                        ix[r, pl.ds(c, 16)] = sv * (2 * HEADS) + q

                    cps = [pltpu.async_copy(xh_h.at[ix.at[i]],
                                            rows.at[pl.ds(i * 128, 128)],
                                            sem1)
                           for i in range(MAC2)]
                    for cp in cps:
                        cp.wait()

                    @pl.loop(0, MAC2 * 8)
                    def _(g):
                        exv = exb[pl.ds(g * 16, 16)]
                        for e in range(16):
                            rr = g * 16 + e
                            rows[rr, :] = rows[rr, :] * exv[e]

                    adds = []
                    for i in range(MAC2):
                        adds.append(pltpu.async_copy(
                            rows.at[pl.ds(i * 128, 128)], acc.at[di.at[i]],
                            sem2, add=True))
                        if len(adds) >= 4:
                            adds.pop(0).wait()
                    for cp in adds:
                        cp.wait()

                plsc.subcore_barrier()
                r0 = sid * ROWS_PER_SUB
                pltpu.sync_copy(acc.at[pl.ds(r0, ROWS_PER_SUB)],
                                out_h.at[q, pl.ds(r0, ROWS_PER_SUB)])
                plsc.subcore_barrier()

    return k(src2, dst2, xh8, ext3)


def kernel(x, edge_attr, params, edge_index, batch):
    npad = E_PAD - E
    pad_src = jnp.arange(npad, dtype=jnp.int32) % N
    pad_dst = N + 104 + (jnp.arange(npad, dtype=jnp.int32) % 64)
    src2 = jnp.concatenate(
        [edge_index[0].astype(jnp.int32), pad_src]).reshape(R_TOT, 128)
    dst2 = jnp.concatenate(
        [edge_index[1].astype(jnp.int32), pad_dst]).reshape(R_TOT, 128)
    batch3 = batch.astype(jnp.int32).reshape(N // NB, 1, NB)

    p = params
    row = lambda v: v.reshape(1, -1)

    egw = p['eg_W'].reshape(LAYERS, HIDDEN // 2, HEADS, C)
    V = jnp.einsum('lkhc,lhc->lkh', egw, p['att_edge'])
    V = jnp.pad(V, ((0, 0), (0, 0), (0, 16 - HEADS)))

    eye4 = jnp.eye(HEADS, dtype=_f32)
    aM = jnp.einsum('lhc,hg->lchg', p['att_src'], eye4).reshape(
        LAYERS, HIDDEN, HEADS)
    aM = jnp.pad(aM, ((0, 0), (0, 0), (0, 16 - HEADS)))
    dM = jnp.einsum('lhc,hg->lchg', p['att_dst'], eye4).reshape(
        LAYERS, HIDDEN, HEADS)
    dM = jnp.pad(dM, ((0, 0), (0, 0), (0, 16 - HEADS)))

    R4 = jnp.kron(eye4, jnp.ones((1, C), _f32))

    h = _node_enc(x, p['ne_W'], row(p['ne_b']), row(p['ne_g']),
                  row(p['ne_beta']))
    ae = _edge_enc(edge_attr, p['ee_W'], row(p['ee_b']), V)

    for l in range(LAYERS):
        xh, asrc16, adst16 = _layer_proj(h, p['gat_W'][l], aM[l], dM[l])
        ae_pk = ae[l].reshape(E_PAD // 8, 128)
        ext3, denp = _sc_pass1(src2, dst2, asrc16, adst16, ae_pk)
        xh8 = xh.reshape(2 * HEADS * N, 16)
        out8 = _sc_pass2(src2, dst2, xh8, ext3)
        h = _layer_norm_res(h, out8, denp, R4,
                            row(p['gat_b'][l]), row(p['ln_g'][l]),
                            row(p['ln_b'][l]))

    g = _pool(h, batch3)
    out = _head_mlp(g, p['f1_W'], row(p['f1_b']), p['f2_W'], row(p['f2_b']),
                    p['c1_W'], row(p['c1_b']), row(p['c2_W'][:, 0]),
                    p['c2_b'].reshape(1, 1))
    return out.reshape(NGRAPH)


# speedup vs baseline: 1.2889x; 1.0594x over previous
import dataclasses
import functools

import jax
import jax.numpy as jnp
from jax import lax
from jax.experimental import pallas as pl
from jax.experimental.pallas import tpu as pltpu
from jax.experimental.pallas import tpu_sc as plsc

N = 50000
E = 800000
NODE_DIM = 12
EDGE_DIM = 4
HIDDEN = 128
HEADS = 4
C = 32
LAYERS = 3
NGRAPH = 64

NB = 1000
EB = 2000
E_PAD = 802816
R_TOT = E_PAD // 128
MAC = 7
MAC2 = 14
NSUB = 16
NWORK = 32
N_PAD = 50176
ROWS_PER_SUB = N_PAD // NSUB
ZR = 112

_f32 = jnp.float32


def _gelu(x):
    return 0.5 * x * (1.0 + lax.erf(x * (2.0 ** -0.5)))


def _ln_rows(h, g, b):
    mu = jnp.mean(h, axis=-1, keepdims=True)
    v = jnp.mean((h - mu) ** 2, axis=-1, keepdims=True)
    return (h - mu) / jnp.sqrt(v + 1e-5) * g + b


def _node_enc(x, W, b, g, beta):
    def body(x_ref, w_ref, b_ref, g_ref, be_ref, o_ref):
        h = jnp.dot(x_ref[...], w_ref[...], preferred_element_type=_f32)
        h = h + b_ref[...]
        o_ref[...] = _gelu(_ln_rows(h, g_ref[...], be_ref[...]))

    return pl.pallas_call(
        body,
        grid=(N // NB,),
        in_specs=[
            pl.BlockSpec((NB, NODE_DIM), lambda i: (i, 0)),
            pl.BlockSpec((NODE_DIM, HIDDEN), lambda i: (0, 0)),
            pl.BlockSpec((1, HIDDEN), lambda i: (0, 0)),
            pl.BlockSpec((1, HIDDEN), lambda i: (0, 0)),
            pl.BlockSpec((1, HIDDEN), lambda i: (0, 0)),
        ],
        out_specs=pl.BlockSpec((NB, HIDDEN), lambda i: (i, 0)),
        out_shape=jax.ShapeDtypeStruct((N, HIDDEN), _f32),
    )(x, W, b, g, beta)


def _edge_enc(edge_attr, eeW, eeb, V):
    def body(ea_ref, w_ref, b_ref, v0_ref, v1_ref, v2_ref, o0, o1, o2):
        t = jnp.dot(ea_ref[...], w_ref[...], preferred_element_type=_f32)
        t = _gelu(t + b_ref[...])
        o0[...] = jnp.dot(t, v0_ref[...], preferred_element_type=_f32)
        o1[...] = jnp.dot(t, v1_ref[...], preferred_element_type=_f32)
        o2[...] = jnp.dot(t, v2_ref[...], preferred_element_type=_f32)

    os = jax.ShapeDtypeStruct((E_PAD, 16), _f32)
    return pl.pallas_call(
        body,
        grid=(E // EB,),
        in_specs=[
            pl.BlockSpec((EB, EDGE_DIM), lambda i: (i, 0)),
            pl.BlockSpec((EDGE_DIM, HIDDEN // 2), lambda i: (0, 0)),
            pl.BlockSpec((1, HIDDEN // 2), lambda i: (0, 0)),
            pl.BlockSpec((HIDDEN // 2, 16), lambda i: (0, 0)),
            pl.BlockSpec((HIDDEN // 2, 16), lambda i: (0, 0)),
            pl.BlockSpec((HIDDEN // 2, 16), lambda i: (0, 0)),
        ],
        out_specs=[pl.BlockSpec((EB, 16), lambda i: (i, 0))] * 3,
        out_shape=[os, os, os],
    )(edge_attr, eeW, eeb, V[0], V[1], V[2])


def _layer_proj(h, W, aM, dM):
    def body(h_ref, w_ref, a_ref, d_ref, xh_ref, as_ref, ad_ref):
        xh = jnp.dot(h_ref[...], w_ref[...], preferred_element_type=_f32)
        xh_ref[...] = xh
        as_ref[...] = jnp.dot(xh, a_ref[...], preferred_element_type=_f32)
        ad_ref[...] = jnp.dot(xh, d_ref[...], preferred_element_type=_f32)

    return pl.pallas_call(
        body,
        grid=(N // NB,),
        in_specs=[
            pl.BlockSpec((NB, HIDDEN), lambda i: (i, 0)),
            pl.BlockSpec((HIDDEN, HIDDEN), lambda i: (0, 0)),
            pl.BlockSpec((HIDDEN, 16), lambda i: (0, 0)),
            pl.BlockSpec((HIDDEN, 16), lambda i: (0, 0)),
        ],
        out_specs=[
            pl.BlockSpec((NB, HIDDEN), lambda i: (i, 0)),
            pl.BlockSpec((NB, 16), lambda i: (i, 0)),
            pl.BlockSpec((NB, 16), lambda i: (i, 0)),
        ],
        out_shape=[
            jax.ShapeDtypeStruct((N, HIDDEN), _f32),
            jax.ShapeDtypeStruct((N, 16), _f32),
            jax.ShapeDtypeStruct((N, 16), _f32),
        ],
    )(h, W, aM, dM)


def _layer_norm_res(h, out, denp, R4, gb, lg, lb):
    def body(h_ref, o_ref, dp_ref, r_ref, gb_ref, g_ref, b_ref, ho_ref):
        den = dp_ref[0, :, 0:4] + dp_ref[1, :, 0:4]
        rec = 1.0 / (den + 1e-16)
        drep = jnp.dot(rec, r_ref[...], preferred_element_type=_f32)
        val = h_ref[...] + o_ref[...] * drep + gb_ref[...]
        ho_ref[...] = _ln_rows(val, g_ref[...], b_ref[...])

    return pl.pallas_call(
        body,
        grid=(N // NB,),
        in_specs=[
            pl.BlockSpec((NB, HIDDEN), lambda i: (i, 0)),
            pl.BlockSpec((NB, HIDDEN), lambda i: (i, 0)),
            pl.BlockSpec((2, NB, 16), lambda i: (0, i, 0)),
            pl.BlockSpec((HEADS, HIDDEN), lambda i: (0, 0)),
            pl.BlockSpec((1, HIDDEN), lambda i: (0, 0)),
            pl.BlockSpec((1, HIDDEN), lambda i: (0, 0)),
            pl.BlockSpec((1, HIDDEN), lambda i: (0, 0)),
        ],
        out_specs=pl.BlockSpec((NB, HIDDEN), lambda i: (i, 0)),
        out_shape=jax.ShapeDtypeStruct((N, HIDDEN), _f32),
    )(h, out, denp, R4, gb, lg, lb)


def _fused_norm_proj(h, out, denp, R4, gb, lg, lb, W, aM, dM):
    def body(h_ref, o_ref, dp_ref, r_ref, gb_ref, g_ref, b_ref,
             w_ref, a_ref, d_ref, ho_ref, xh_ref, as_ref, ad_ref):
        den = dp_ref[0, :, 0:4] + dp_ref[1, :, 0:4]
        rec = 1.0 / (den + 1e-16)
        drep = jnp.dot(rec, r_ref[...], preferred_element_type=_f32)
        val = h_ref[...] + o_ref[...] * drep + gb_ref[...]
        hn = _ln_rows(val, g_ref[...], b_ref[...])
        ho_ref[...] = hn
        xh = jnp.dot(hn, w_ref[...], preferred_element_type=_f32)
        xh_ref[...] = xh
        as_ref[...] = jnp.dot(xh, a_ref[...], preferred_element_type=_f32)
        ad_ref[...] = jnp.dot(xh, d_ref[...], preferred_element_type=_f32)

    return pl.pallas_call(
        body,
        grid=(N // NB,),
        in_specs=[
            pl.BlockSpec((NB, HIDDEN), lambda i: (i, 0)),
            pl.BlockSpec((NB, HIDDEN), lambda i: (i, 0)),
            pl.BlockSpec((2, NB, 16), lambda i: (0, i, 0)),
            pl.BlockSpec((HEADS, HIDDEN), lambda i: (0, 0)),
            pl.BlockSpec((1, HIDDEN), lambda i: (0, 0)),
            pl.BlockSpec((1, HIDDEN), lambda i: (0, 0)),
            pl.BlockSpec((1, HIDDEN), lambda i: (0, 0)),
            pl.BlockSpec((HIDDEN, HIDDEN), lambda i: (0, 0)),
            pl.BlockSpec((HIDDEN, 16), lambda i: (0, 0)),
            pl.BlockSpec((HIDDEN, 16), lambda i: (0, 0)),
        ],
        out_specs=[
            pl.BlockSpec((NB, HIDDEN), lambda i: (i, 0)),
            pl.BlockSpec((NB, HIDDEN), lambda i: (i, 0)),
            pl.BlockSpec((NB, 16), lambda i: (i, 0)),
            pl.BlockSpec((NB, 16), lambda i: (i, 0)),
        ],
        out_shape=[
            jax.ShapeDtypeStruct((N, HIDDEN), _f32),
            jax.ShapeDtypeStruct((N, HIDDEN), _f32),
            jax.ShapeDtypeStruct((N, 16), _f32),
            jax.ShapeDtypeStruct((N, 16), _f32),
        ],
    )(h, out, denp, R4, gb, lg, lb, W, aM, dM)


def _pool(h, batch3):
    def body(h_ref, b_ref, o_ref, s_ref, m_ref, c_ref):
        ones = jnp.ones((NB, HIDDEN), _f32)
        i = pl.program_id(0)

        @pl.when(i == 0)
        def _():
            s_ref[...] = jnp.zeros((NGRAPH, HIDDEN), _f32)
            c_ref[...] = jnp.zeros((NGRAPH, HIDDEN), _f32)
            m_ref[...] = jnp.full((NGRAPH, HIDDEN), -jnp.inf, _f32)

        hb = h_ref[...]
        b = b_ref[0, 0, :]
        oh = (b[:, None] == lax.broadcasted_iota(
            jnp.int32, (NB, NGRAPH), 1)).astype(_f32)
        dn = (((0,), (0,)), ((), ()))
        s_ref[...] += lax.dot_general(oh, hb, dn,
                                      preferred_element_type=_f32)
        c_ref[...] += lax.dot_general(oh, ones, dn,
                                      preferred_element_type=_f32)
        mrows = [
            jnp.max(jnp.where(b[:, None] == g, hb, -jnp.inf), axis=0,
                    keepdims=True)
            for g in range(NGRAPH)
        ]
        m_ref[...] = jnp.maximum(m_ref[...], jnp.concatenate(mrows, axis=0))

        cnt = c_ref[...]
        mean = s_ref[...] / jnp.maximum(cnt, 1.0)
        mx = jnp.where(cnt > 0.0, m_ref[...], 0.0)
        o_ref[...] = jnp.concatenate([mean, mx], axis=-1)

    return pl.pallas_call(
        body,
        grid=(N // NB,),
        in_specs=[
            pl.BlockSpec((NB, HIDDEN), lambda i: (i, 0)),
            pl.BlockSpec((1, 1, NB), lambda i: (i, 0, 0)),
        ],
        out_specs=pl.BlockSpec((NGRAPH, 2 * HIDDEN), lambda i: (0, 0)),
        out_shape=jax.ShapeDtypeStruct((NGRAPH, 2 * HIDDEN), _f32),
        scratch_shapes=[
            pltpu.VMEM((NGRAPH, HIDDEN), _f32),
            pltpu.VMEM((NGRAPH, HIDDEN), _f32),
            pltpu.VMEM((NGRAPH, HIDDEN), _f32),
        ],
    )(h, batch3)


def _head_mlp(g, f1W, f1b, f2W, f2b, c1W, c1b, c2r, c2b):
    def body(g_ref, w1, b1, w2, b2, w3, b3, w4, b4, o_ref):
        x = _gelu(jnp.dot(g_ref[...], w1[...],
                          preferred_element_type=_f32) + b1[...])
        x = _gelu(jnp.dot(x, w2[...],
                          preferred_element_type=_f32) + b2[...])
        x = _gelu(jnp.dot(x, w3[...],
                          preferred_element_type=_f32) + b3[...])
        logits = jnp.sum(x * w4[...], axis=1, keepdims=True) + b4[...]
        o_ref[...] = 1.0 / (1.0 + jnp.exp(-logits))

    full = lambda s: pl.BlockSpec(s, lambda: tuple(0 for _ in s))
    return pl.pallas_call(
        body,
        in_specs=[full((NGRAPH, 2 * HIDDEN)),
                  full((2 * HIDDEN, HIDDEN)), full((1, HIDDEN)),
                  full((HIDDEN, HIDDEN // 2)), full((1, HIDDEN // 2)),
                  full((HIDDEN // 2, 64)), full((1, 64)),
                  full((1, 64)), full((1, 1))],
        out_specs=full((NGRAPH, 1)),
        out_shape=jax.ShapeDtypeStruct((NGRAPH, 1), _f32),
    )(g, f1W, f1b, f2W, f2b, c1W, c1b, c2r, c2b)


_MESH = plsc.VectorSubcoreMesh(core_axis_name="c", subcore_axis_name="s")
_SC_PARAMS = pltpu.CompilerParams()
if "needs_layout_passes" in pltpu.CompilerParams.__dataclass_fields__:
    _SC_PARAMS = dataclasses.replace(_SC_PARAMS, needs_layout_passes=False)
if "use_tc_tiling_on_sc" in pltpu.CompilerParams.__dataclass_fields__:
    _SC_PARAMS = dataclasses.replace(_SC_PARAMS, use_tc_tiling_on_sc=False)

RPW1 = R_TOT // NWORK
RPS2 = R_TOT // NSUB


def _sc_pass1(src2, dst2, asrc16, adst16, ae16):
    @functools.partial(
        pl.kernel,
        mesh=_MESH,
        compiler_params=_SC_PARAMS,
        out_type=[jax.ShapeDtypeStruct((HEADS * E_PAD,), _f32),
                  jax.ShapeDtypeStruct((2, N_PAD, 16), _f32)],
        scratch_types=[
            pltpu.VMEM((MAC, 128), jnp.int32),
            pltpu.VMEM((MAC, 128), jnp.int32),
            pltpu.VMEM((MAC * 128, 16), _f32),
            pltpu.VMEM((MAC * 128, 16), _f32),
            pltpu.VMEM((MAC * 16, 128), _f32),
            pltpu.VMEM((MAC * 128, 16), _f32),
            pltpu.VMEM((HEADS * MAC * 128,), _f32),
            pltpu.VMEM((ZR, 16), _f32),
            pltpu.VMEM_SHARED((N_PAD, 16), _f32),
            pltpu.SemaphoreType.DMA,
            pltpu.SemaphoreType.DMA,
        ],
    )
    def k(src_h, dst_h, as_h, ad_h, ae_h, ext_h, denp_h,
          si, di, ar, dr, er, xr, xtb, zb, dacc, sem1, sem2):
        cid = lax.axis_index("c")
        sid = lax.axis_index("s")

        @pl.loop(0, ZR)
        def _(i):
            zb[i, :] = jnp.zeros((16,), _f32)

        @pl.loop(0, ROWS_PER_SUB // ZR)
        def _(j):
            pltpu.sync_copy(zb, dacc.at[pl.ds(sid * ROWS_PER_SUB + j * ZR, ZR)])

        plsc.subcore_barrier()

        row0 = (cid * NSUB + sid) * RPW1
        lane = lax.iota(jnp.int32, 16)
        lane_lt4 = lane < HEADS
        scat_pat = lane * (MAC * 128)

        @pl.loop(0, RPW1 // MAC)
        def _(ci):
            rowb = row0 + ci * MAC
            cps = [pltpu.async_copy(src_h.at[pl.ds(rowb, MAC)], si, sem1),
                   pltpu.async_copy(dst_h.at[pl.ds(rowb, MAC)], di, sem1),
                   pltpu.async_copy(ae_h.at[pl.ds(rowb * 16, MAC * 16)],
                                    er, sem1)]
            for cp in cps:
                cp.wait()
            cps = []
            for i in range(MAC):
                cps.append(pltpu.async_copy(
                    as_h.at[si.at[i]], ar.at[pl.ds(i * 128, 128)], sem1))
                cps.append(pltpu.async_copy(
                    ad_h.at[di.at[i]], dr.at[pl.ds(i * 128, 128)], sem1))
            for cp in cps:
                cp.wait()

            @pl.loop(0, MAC * 128)
            def _(e):
                srow = ar[e, :] + dr[e, :] + er[e >> 3, pl.ds((e & 7) * 16, 16)]
                srow = jnp.maximum(srow, 0.2 * srow)
                exr = jnp.exp(srow)
                xr[e, :] = exr
                plsc.store_scatter(xtb, [scat_pat + e], exr, mask=lane_lt4)

            adds = []
            for i in range(MAC):
                adds.append(pltpu.async_copy(
                    xr.at[pl.ds(i * 128, 128)], dacc.at[di.at[i]], sem2,
                    add=True))
                if len(adds) >= 4:
                    adds.pop(0).wait()
            for cp in adds:
                cp.wait()
            cps = [pltpu.async_copy(
                xtb.at[pl.ds(hh * MAC * 128, MAC * 128)],
                ext_h.at[pl.ds(hh * E_PAD + rowb * 128, MAC * 128)], sem2)
                   for hh in range(HEADS)]
            for cp in cps:
                cp.wait()

        plsc.subcore_barrier()
        r0 = sid * ROWS_PER_SUB
        pltpu.sync_copy(dacc.at[pl.ds(r0, ROWS_PER_SUB)],
                        denp_h.at[cid, pl.ds(r0, ROWS_PER_SUB)])

    return k(src2, dst2, asrc16, adst16, ae16)


def _sc_pass2(src2, dst2, xh8, ext3):
    @functools.partial(
        pl.kernel,
        mesh=_MESH,
        compiler_params=_SC_PARAMS,
        out_type=jax.ShapeDtypeStruct((N_PAD, HIDDEN), _f32),
        scratch_types=[
            pltpu.VMEM((MAC2, 128), jnp.int32),
            pltpu.VMEM((MAC2, 128), jnp.int32),
            pltpu.VMEM((MAC2, 128), jnp.int32),
            pltpu.VMEM((MAC2 * 128,), _f32),
            pltpu.VMEM((MAC2 * 128, 16), _f32),
            pltpu.VMEM((ZR, 16), _f32),
            pltpu.VMEM_SHARED((N_PAD, 16), _f32),
            pltpu.SemaphoreType.DMA,
            pltpu.SemaphoreType.DMA,
        ],
    )
    def k(src_h, dst_h, xh_h, ext_h, out_h,
          si, di, ix, exb, rows, zb, acc, sem1, sem2):
        cid = lax.axis_index("c")
        sid = lax.axis_index("s")

        @pl.loop(0, ZR)
        def _(i):
            zb[i, :] = jnp.zeros((16,), _f32)

        for hp in range(2):
            for half in range(2):
                q = cid * 4 + hp * 2 + half
                head = cid * 2 + hp

                @pl.loop(0, ROWS_PER_SUB // ZR)
                def _(j):
                    pltpu.sync_copy(
                        zb, acc.at[pl.ds(sid * ROWS_PER_SUB + j * ZR, ZR)])

                plsc.subcore_barrier()

                row0 = sid * RPS2

                @pl.loop(0, RPS2 // MAC2)
                def _(ci):
                    rowb = row0 + ci * MAC2
                    cps = [
                        pltpu.async_copy(src_h.at[pl.ds(rowb, MAC2)], si,
                                         sem1),
                        pltpu.async_copy(dst_h.at[pl.ds(rowb, MAC2)], di,
                                         sem1),
                        pltpu.async_copy(
                            ext_h.at[pl.ds(head * E_PAD + rowb * 128,
                                           MAC2 * 128)], exb, sem1),
                    ]
                    for cp in cps:
                        cp.wait()

                    @pl.loop(0, MAC2 * 8)
                    def _(g):
                        r = g >> 3
                        c = (g & 7) * 16
                        sv = si[r, pl.ds(c, 16)]
                        ix[r, pl.ds(c, 16)] = sv * (2 * HEADS) + q

                    cps = [pltpu.async_copy(xh_h.at[ix.at[i]],
                                            rows.at[pl.ds(i * 128, 128)],
                                            sem1)
                           for i in range(MAC2)]
                    for cp in cps:
                        cp.wait()

                    @pl.loop(0, MAC2 * 8)
                    def _(g):
                        exv = exb[pl.ds(g * 16, 16)]
                        for e in range(16):
                            rr = g * 16 + e
                            rows[rr, :] = rows[rr, :] * exv[e]

                    adds = []
                    for i in range(MAC2):
                        adds.append(pltpu.async_copy(
                            rows.at[pl.ds(i * 128, 128)], acc.at[di.at[i]],
                            sem2, add=True))
                        if len(adds) >= 4:
                            adds.pop(0).wait()
                    for cp in adds:
                        cp.wait()

                plsc.subcore_barrier()
                r0 = sid * ROWS_PER_SUB
                pltpu.sync_copy(acc.at[pl.ds(r0, ROWS_PER_SUB)],
                                out_h.at[pl.ds(r0, ROWS_PER_SUB),
                                         pl.ds(q * 16, 16)])
                plsc.subcore_barrier()

    return k(src2, dst2, xh8, ext3)


def kernel(x, edge_attr, params, edge_index, batch):
    npad = E_PAD - E
    pad_src = jnp.arange(npad, dtype=jnp.int32) % N
    pad_dst = N + 104 + (jnp.arange(npad, dtype=jnp.int32) % 64)
    src2 = jnp.concatenate(
        [edge_index[0].astype(jnp.int32), pad_src]).reshape(R_TOT, 128)
    dst2 = jnp.concatenate(
        [edge_index[1].astype(jnp.int32), pad_dst]).reshape(R_TOT, 128)
    batch3 = batch.astype(jnp.int32).reshape(N // NB, 1, NB)

    p = params
    row = lambda v: v.reshape(1, -1)

    egw = p['eg_W'].reshape(LAYERS, HIDDEN // 2, HEADS, C)
    V = jnp.einsum('lkhc,lhc->lkh', egw, p['att_edge'])
    V = jnp.pad(V, ((0, 0), (0, 0), (0, 16 - HEADS)))

    eye4 = jnp.eye(HEADS, dtype=_f32)
    aM = jnp.einsum('lhc,hg->lchg', p['att_src'], eye4).reshape(
        LAYERS, HIDDEN, HEADS)
    aM = jnp.pad(aM, ((0, 0), (0, 0), (0, 16 - HEADS)))
    dM = jnp.einsum('lhc,hg->lchg', p['att_dst'], eye4).reshape(
        LAYERS, HIDDEN, HEADS)
    dM = jnp.pad(dM, ((0, 0), (0, 0), (0, 16 - HEADS)))

    R4 = jnp.kron(eye4, jnp.ones((1, C), _f32))

    h = _node_enc(x, p['ne_W'], row(p['ne_b']), row(p['ne_g']),
                  row(p['ne_beta']))
    ae = _edge_enc(edge_attr, p['ee_W'], row(p['ee_b']), V)

    xh, asrc16, adst16 = _layer_proj(h, p['gat_W'][0], aM[0], dM[0])
    for l in range(LAYERS):
        ae_pk = ae[l].reshape(E_PAD // 8, 128)
        ext3, denp = _sc_pass1(src2, dst2, asrc16, adst16, ae_pk)
        xh8 = xh.reshape(2 * HEADS * N, 16)
        out = _sc_pass2(src2, dst2, xh8, ext3)[:N]
        if l + 1 < LAYERS:
            h, xh, asrc16, adst16 = _fused_norm_proj(
                h, out, denp, R4, row(p['gat_b'][l]), row(p['ln_g'][l]),
                row(p['ln_b'][l]), p['gat_W'][l + 1], aM[l + 1], dM[l + 1])
        else:
            h = _layer_norm_res(h, out, denp, R4, row(p['gat_b'][l]),
                                row(p['ln_g'][l]), row(p['ln_b'][l]))
        g = _pool(h, batch3)
    out = _head_mlp(g, p['f1_W'], row(p['f1_b']), p['f2_W'], row(p['f2_b']),
                    p['c1_W'], row(p['c1_b']), row(p['c2_W'][:, 0]),
                    p['c2_b'].reshape(1, 1))
    return out.reshape(NGRAPH)


# speedup vs baseline: 1.4205x; 1.1021x over previous
import dataclasses
import functools

import jax
import jax.numpy as jnp
from jax import lax
from jax.experimental import pallas as pl
from jax.experimental.pallas import tpu as pltpu
from jax.experimental.pallas import tpu_sc as plsc

N = 50000
E = 800000
NODE_DIM = 12
EDGE_DIM = 4
HIDDEN = 128
HEADS = 4
C = 32
LAYERS = 3
NGRAPH = 64

NB = 1000
EB = 2000
E_PAD = 802816
R_TOT = E_PAD // 128
MAC = 7
MAC2 = 14
NSUB = 16
NWORK = 32
N_PAD = 50176
ROWS_PER_SUB = N_PAD // NSUB
ZR = 112

_f32 = jnp.float32


def _gelu(x):
    return 0.5 * x * (1.0 + lax.erf(x * (2.0 ** -0.5)))


def _ln_rows(h, g, b):
    mu = jnp.mean(h, axis=-1, keepdims=True)
    v = jnp.mean((h - mu) ** 2, axis=-1, keepdims=True)
    return (h - mu) / jnp.sqrt(v + 1e-5) * g + b


def _node_enc(x, W, b, g, beta):
    def body(x_ref, w_ref, b_ref, g_ref, be_ref, o_ref):
        h = jnp.dot(x_ref[...], w_ref[...], preferred_element_type=_f32)
        h = h + b_ref[...]
        o_ref[...] = _gelu(_ln_rows(h, g_ref[...], be_ref[...]))

    return pl.pallas_call(
        body,
        grid=(N // NB,),
        in_specs=[
            pl.BlockSpec((NB, NODE_DIM), lambda i: (i, 0)),
            pl.BlockSpec((NODE_DIM, HIDDEN), lambda i: (0, 0)),
            pl.BlockSpec((1, HIDDEN), lambda i: (0, 0)),
            pl.BlockSpec((1, HIDDEN), lambda i: (0, 0)),
            pl.BlockSpec((1, HIDDEN), lambda i: (0, 0)),
        ],
        out_specs=pl.BlockSpec((NB, HIDDEN), lambda i: (i, 0)),
        out_shape=jax.ShapeDtypeStruct((N, HIDDEN), _f32),
    )(x, W, b, g, beta)


def _edge_enc(edge_attr, eeW, eeb, V):
    def body(ea_ref, w_ref, b_ref, v0_ref, v1_ref, v2_ref, o0, o1, o2):
        t = jnp.dot(ea_ref[...], w_ref[...], preferred_element_type=_f32)
        t = _gelu(t + b_ref[...])
        o0[...] = jnp.dot(t, v0_ref[...], preferred_element_type=_f32)
        o1[...] = jnp.dot(t, v1_ref[...], preferred_element_type=_f32)
        o2[...] = jnp.dot(t, v2_ref[...], preferred_element_type=_f32)

    os = jax.ShapeDtypeStruct((E_PAD, 16), _f32)
    return pl.pallas_call(
        body,
        grid=(E // EB,),
        in_specs=[
            pl.BlockSpec((EB, EDGE_DIM), lambda i: (i, 0)),
            pl.BlockSpec((EDGE_DIM, HIDDEN // 2), lambda i: (0, 0)),
            pl.BlockSpec((1, HIDDEN // 2), lambda i: (0, 0)),
            pl.BlockSpec((HIDDEN // 2, 16), lambda i: (0, 0)),
            pl.BlockSpec((HIDDEN // 2, 16), lambda i: (0, 0)),
            pl.BlockSpec((HIDDEN // 2, 16), lambda i: (0, 0)),
        ],
        out_specs=[pl.BlockSpec((EB, 16), lambda i: (i, 0))] * 3,
        out_shape=[os, os, os],
    )(edge_attr, eeW, eeb, V[0], V[1], V[2])


def _layer_proj(h, W, aM, dM):
    def body(h_ref, w_ref, a_ref, d_ref, xh_ref, as_ref, ad_ref):
        xh = jnp.dot(h_ref[...], w_ref[...], preferred_element_type=_f32)
        xh_ref[...] = xh
        as_ref[...] = jnp.dot(xh, a_ref[...], preferred_element_type=_f32)
        ad_ref[...] = jnp.dot(xh, d_ref[...], preferred_element_type=_f32)

    return pl.pallas_call(
        body,
        grid=(N // NB,),
        in_specs=[
            pl.BlockSpec((NB, HIDDEN), lambda i: (i, 0)),
            pl.BlockSpec((HIDDEN, HIDDEN), lambda i: (0, 0)),
            pl.BlockSpec((HIDDEN, 16), lambda i: (0, 0)),
            pl.BlockSpec((HIDDEN, 16), lambda i: (0, 0)),
        ],
        out_specs=[
            pl.BlockSpec((NB, HIDDEN), lambda i: (i, 0)),
            pl.BlockSpec((NB, 16), lambda i: (i, 0)),
            pl.BlockSpec((NB, 16), lambda i: (i, 0)),
        ],
        out_shape=[
            jax.ShapeDtypeStruct((N, HIDDEN), _f32),
            jax.ShapeDtypeStruct((N, 16), _f32),
            jax.ShapeDtypeStruct((N, 16), _f32),
        ],
    )(h, W, aM, dM)


def _layer_norm_res(h, out, denp, R4, gb, lg, lb):
    def body(h_ref, o_ref, dp_ref, r_ref, gb_ref, g_ref, b_ref, ho_ref):
        den = dp_ref[0, :, 0:4] + dp_ref[1, :, 0:4]
        rec = 1.0 / (den + 1e-16)
        drep = jnp.dot(rec, r_ref[...], preferred_element_type=_f32)
        val = h_ref[...] + o_ref[...] * drep + gb_ref[...]
        ho_ref[...] = _ln_rows(val, g_ref[...], b_ref[...])

    return pl.pallas_call(
        body,
        grid=(N // NB,),
        in_specs=[
            pl.BlockSpec((NB, HIDDEN), lambda i: (i, 0)),
            pl.BlockSpec((NB, HIDDEN), lambda i: (i, 0)),
            pl.BlockSpec((2, NB, 16), lambda i: (0, i, 0)),
            pl.BlockSpec((HEADS, HIDDEN), lambda i: (0, 0)),
            pl.BlockSpec((1, HIDDEN), lambda i: (0, 0)),
            pl.BlockSpec((1, HIDDEN), lambda i: (0, 0)),
            pl.BlockSpec((1, HIDDEN), lambda i: (0, 0)),
        ],
        out_specs=pl.BlockSpec((NB, HIDDEN), lambda i: (i, 0)),
        out_shape=jax.ShapeDtypeStruct((N, HIDDEN), _f32),
    )(h, out, denp, R4, gb, lg, lb)


def _fused_norm_proj(h, out, denp, R4, gb, lg, lb, W, aM, dM):
    def body(h_ref, o_ref, dp_ref, r_ref, gb_ref, g_ref, b_ref,
             w_ref, a_ref, d_ref, ho_ref, xh_ref, as_ref, ad_ref):
        den = dp_ref[0, :, 0:4] + dp_ref[1, :, 0:4]
        rec = 1.0 / (den + 1e-16)
        drep = jnp.dot(rec, r_ref[...], preferred_element_type=_f32)
        val = h_ref[...] + o_ref[...] * drep + gb_ref[...]
        hn = _ln_rows(val, g_ref[...], b_ref[...])
        ho_ref[...] = hn
        xh = jnp.dot(hn, w_ref[...], preferred_element_type=_f32)
        xh_ref[...] = xh
        as_ref[...] = jnp.dot(xh, a_ref[...], preferred_element_type=_f32)
        ad_ref[...] = jnp.dot(xh, d_ref[...], preferred_element_type=_f32)

    return pl.pallas_call(
        body,
        grid=(N // NB,),
        in_specs=[
            pl.BlockSpec((NB, HIDDEN), lambda i: (i, 0)),
            pl.BlockSpec((NB, HIDDEN), lambda i: (i, 0)),
            pl.BlockSpec((2, NB, 16), lambda i: (0, i, 0)),
            pl.BlockSpec((HEADS, HIDDEN), lambda i: (0, 0)),
            pl.BlockSpec((1, HIDDEN), lambda i: (0, 0)),
            pl.BlockSpec((1, HIDDEN), lambda i: (0, 0)),
            pl.BlockSpec((1, HIDDEN), lambda i: (0, 0)),
            pl.BlockSpec((HIDDEN, HIDDEN), lambda i: (0, 0)),
            pl.BlockSpec((HIDDEN, 16), lambda i: (0, 0)),
            pl.BlockSpec((HIDDEN, 16), lambda i: (0, 0)),
        ],
        out_specs=[
            pl.BlockSpec((NB, HIDDEN), lambda i: (i, 0)),
            pl.BlockSpec((NB, HIDDEN), lambda i: (i, 0)),
            pl.BlockSpec((NB, 16), lambda i: (i, 0)),
            pl.BlockSpec((NB, 16), lambda i: (i, 0)),
        ],
        out_shape=[
            jax.ShapeDtypeStruct((N, HIDDEN), _f32),
            jax.ShapeDtypeStruct((N, HIDDEN), _f32),
            jax.ShapeDtypeStruct((N, 16), _f32),
            jax.ShapeDtypeStruct((N, 16), _f32),
        ],
    )(h, out, denp, R4, gb, lg, lb, W, aM, dM)


def _pool(h, batch3):
    def body(h_ref, b_ref, o_ref, s_ref, m_ref, c_ref):
        ones = jnp.ones((NB, HIDDEN), _f32)
        i = pl.program_id(0)

        @pl.when(i == 0)
        def _():
            s_ref[...] = jnp.zeros((NGRAPH, HIDDEN), _f32)
            c_ref[...] = jnp.zeros((NGRAPH, HIDDEN), _f32)
            m_ref[...] = jnp.full((NGRAPH, HIDDEN), -jnp.inf, _f32)

        hb = h_ref[...]
        b = b_ref[0, 0, :]
        oh = (b[:, None] == lax.broadcasted_iota(
            jnp.int32, (NB, NGRAPH), 1)).astype(_f32)
        dn = (((0,), (0,)), ((), ()))
        s_ref[...] += lax.dot_general(oh, hb, dn,
                                      preferred_element_type=_f32)
        c_ref[...] += lax.dot_general(oh, ones, dn,
                                      preferred_element_type=_f32)
        mrows = [
            jnp.max(jnp.where(b[:, None] == g, hb, -jnp.inf), axis=0,
                    keepdims=True)
            for g in range(NGRAPH)
        ]
        m_ref[...] = jnp.maximum(m_ref[...], jnp.concatenate(mrows, axis=0))

        cnt = c_ref[...]
        mean = s_ref[...] / jnp.maximum(cnt, 1.0)
        mx = jnp.where(cnt > 0.0, m_ref[...], 0.0)
        o_ref[...] = jnp.concatenate([mean, mx], axis=-1)

    return pl.pallas_call(
        body,
        grid=(N // NB,),
        in_specs=[
            pl.BlockSpec((NB, HIDDEN), lambda i: (i, 0)),
            pl.BlockSpec((1, 1, NB), lambda i: (i, 0, 0)),
        ],
        out_specs=pl.BlockSpec((NGRAPH, 2 * HIDDEN), lambda i: (0, 0)),
        out_shape=jax.ShapeDtypeStruct((NGRAPH, 2 * HIDDEN), _f32),
        scratch_shapes=[
            pltpu.VMEM((NGRAPH, HIDDEN), _f32),
            pltpu.VMEM((NGRAPH, HIDDEN), _f32),
            pltpu.VMEM((NGRAPH, HIDDEN), _f32),
        ],
    )(h, batch3)


def _head_mlp(g, f1W, f1b, f2W, f2b, c1W, c1b, c2r, c2b):
    def body(g_ref, w1, b1, w2, b2, w3, b3, w4, b4, o_ref):
        x = _gelu(jnp.dot(g_ref[...], w1[...],
                          preferred_element_type=_f32) + b1[...])
        x = _gelu(jnp.dot(x, w2[...],
                          preferred_element_type=_f32) + b2[...])
        x = _gelu(jnp.dot(x, w3[...],
                          preferred_element_type=_f32) + b3[...])
        logits = jnp.sum(x * w4[...], axis=1, keepdims=True) + b4[...]
        o_ref[...] = 1.0 / (1.0 + jnp.exp(-logits))

    full = lambda s: pl.BlockSpec(s, lambda: tuple(0 for _ in s))
    return pl.pallas_call(
        body,
        in_specs=[full((NGRAPH, 2 * HIDDEN)),
                  full((2 * HIDDEN, HIDDEN)), full((1, HIDDEN)),
                  full((HIDDEN, HIDDEN // 2)), full((1, HIDDEN // 2)),
                  full((HIDDEN // 2, 64)), full((1, 64)),
                  full((1, 64)), full((1, 1))],
        out_specs=full((NGRAPH, 1)),
        out_shape=jax.ShapeDtypeStruct((NGRAPH, 1), _f32),
    )(g, f1W, f1b, f2W, f2b, c1W, c1b, c2r, c2b)


_MESH = plsc.VectorSubcoreMesh(core_axis_name="c", subcore_axis_name="s")
_SC_PARAMS = pltpu.CompilerParams()
if "needs_layout_passes" in pltpu.CompilerParams.__dataclass_fields__:
    _SC_PARAMS = dataclasses.replace(_SC_PARAMS, needs_layout_passes=False)
if "use_tc_tiling_on_sc" in pltpu.CompilerParams.__dataclass_fields__:
    _SC_PARAMS = dataclasses.replace(_SC_PARAMS, use_tc_tiling_on_sc=False)

RPW1 = R_TOT // NWORK
RPS2 = R_TOT // NSUB


def _sc_pass1(src2, dst2, asrc16, adst16, ae16):
    @functools.partial(
        pl.kernel,
        mesh=_MESH,
        compiler_params=_SC_PARAMS,
        out_type=[jax.ShapeDtypeStruct((HEADS * E_PAD,), _f32),
                  jax.ShapeDtypeStruct((2, N_PAD, 16), _f32)],
        scratch_types=[
            pltpu.VMEM((MAC, 128), jnp.int32),
            pltpu.VMEM((MAC, 128), jnp.int32),
            pltpu.VMEM((MAC * 128, 16), _f32),
            pltpu.VMEM((MAC * 128, 16), _f32),
            pltpu.VMEM((MAC * 16, 128), _f32),
            pltpu.VMEM((MAC * 128, 16), _f32),
            pltpu.VMEM((HEADS * MAC * 128,), _f32),
            pltpu.VMEM((ZR, 16), _f32),
            pltpu.VMEM_SHARED((N_PAD, 16), _f32),
            pltpu.SemaphoreType.DMA,
            pltpu.SemaphoreType.DMA,
        ],
    )
    def k(src_h, dst_h, as_h, ad_h, ae_h, ext_h, denp_h,
          si, di, ar, dr, er, xr, xtb, zb, dacc, sem1, sem2):
        cid = lax.axis_index("c")
        sid = lax.axis_index("s")

        @pl.loop(0, ZR)
        def _(i):
            zb[i, :] = jnp.zeros((16,), _f32)

        @pl.loop(0, ROWS_PER_SUB // ZR)
        def _(j):
            pltpu.sync_copy(zb, dacc.at[pl.ds(sid * ROWS_PER_SUB + j * ZR, ZR)])

        plsc.subcore_barrier()

        row0 = (cid * NSUB + sid) * RPW1
        lane = lax.iota(jnp.int32, 16)
        lane_lt4 = lane < HEADS
        scat_pat = lane * (MAC * 128)

        @pl.loop(0, RPW1 // MAC)
        def _(ci):
            rowb = row0 + ci * MAC
            cps = [pltpu.async_copy(src_h.at[pl.ds(rowb, MAC)], si, sem1),
                   pltpu.async_copy(dst_h.at[pl.ds(rowb, MAC)], di, sem1),
                   pltpu.async_copy(ae_h.at[pl.ds(rowb * 16, MAC * 16)],
                                    er, sem1)]
            for cp in cps:
                cp.wait()
            cps = []
            for i in range(MAC):
                cps.append(pltpu.async_copy(
                    as_h.at[si.at[i]], ar.at[pl.ds(i * 128, 128)], sem1))
                cps.append(pltpu.async_copy(
                    ad_h.at[di.at[i]], dr.at[pl.ds(i * 128, 128)], sem1))
            for cp in cps:
                cp.wait()

            @pl.loop(0, MAC * 128)
            def _(e):
                srow = ar[e, :] + dr[e, :] + er[e >> 3, pl.ds((e & 7) * 16, 16)]
                srow = jnp.maximum(srow, 0.2 * srow)
                exr = jnp.exp(srow)
                xr[e, :] = exr
                plsc.store_scatter(xtb, [scat_pat + e], exr, mask=lane_lt4)

            adds = []
            for i in range(MAC):
                adds.append(pltpu.async_copy(
                    xr.at[pl.ds(i * 128, 128)], dacc.at[di.at[i]], sem2,
                    add=True))
                if len(adds) >= 4:
                    adds.pop(0).wait()
            for cp in adds:
                cp.wait()
            cps = [pltpu.async_copy(
                xtb.at[pl.ds(hh * MAC * 128, MAC * 128)],
                ext_h.at[pl.ds(hh * E_PAD + rowb * 128, MAC * 128)], sem2)
                   for hh in range(HEADS)]
            for cp in cps:
                cp.wait()

        plsc.subcore_barrier()
        r0 = sid * ROWS_PER_SUB
        pltpu.sync_copy(dacc.at[pl.ds(r0, ROWS_PER_SUB)],
                        denp_h.at[cid, pl.ds(r0, ROWS_PER_SUB)])

    return k(src2, dst2, asrc16, adst16, ae16)


def _sc_pass2(src2, dst2, xh8, ext3):
    @functools.partial(
        pl.kernel,
        mesh=_MESH,
        compiler_params=_SC_PARAMS,
        out_type=jax.ShapeDtypeStruct((N_PAD, HIDDEN), _f32),
        scratch_types=[
            pltpu.VMEM((2, MAC2, 128), jnp.int32),
            pltpu.VMEM((2, MAC2, 128), jnp.int32),
            pltpu.VMEM((2, MAC2, 128), jnp.int32),
            pltpu.VMEM((2, MAC2 * 128), _f32),
            pltpu.VMEM((2, MAC2 * 128, 16), _f32),
            pltpu.VMEM((ZR, 16), _f32),
            pltpu.VMEM_SHARED((N_PAD, 16), _f32),
            pltpu.SemaphoreType.DMA,
            pltpu.SemaphoreType.DMA,
            pltpu.SemaphoreType.DMA,
            pltpu.SemaphoreType.DMA,
        ],
    )
    def k(src_h, dst_h, xh_h, ext_h, out_h,
          si, di, ix, exb, rows, zb, acc, semA, semB, semG, sem2):
        cid = lax.axis_index("c")
        sid = lax.axis_index("s")

        @pl.loop(0, ZR)
        def _(i):
            zb[i, :] = jnp.zeros((16,), _f32)

        for hp in range(2):
            for half in range(2):
                q = cid * 4 + hp * 2 + half
                head = cid * 2 + hp

                @pl.loop(0, ROWS_PER_SUB // ZR)
                def _(j):
                    pltpu.sync_copy(
                        zb, acc.at[pl.ds(sid * ROWS_PER_SUB + j * ZR, ZR)])

                plsc.subcore_barrier()

                row0 = sid * RPS2

                @pl.loop(0, RPS2 // (2 * MAC2))
                def _(cj):
                    rowb0 = row0 + cj * (2 * MAC2)

                    ins = []
                    for p_, sem_ in ((0, semA), (1, semB)):
                        rowb = rowb0 + p_ * MAC2
                        ins.append([
                            pltpu.async_copy(src_h.at[pl.ds(rowb, MAC2)],
                                             si.at[p_], sem_),
                            pltpu.async_copy(dst_h.at[pl.ds(rowb, MAC2)],
                                             di.at[p_], sem_),
                            pltpu.async_copy(
                                ext_h.at[pl.ds(head * E_PAD + rowb * 128,
                                               MAC2 * 128)],
                                exb.at[p_], sem_),
                        ])

                    gth = []
                    for p_ in (0, 1):
                        for cp in ins[p_]:
                            cp.wait()

                        @pl.loop(0, MAC2 * 8)
                        def _(g):
                            r = g >> 3
                            c = (g & 7) * 16
                            sv = si[p_, r, pl.ds(c, 16)]
                            ix[p_, r, pl.ds(c, 16)] = sv * (2 * HEADS) + q

                        gth.append([pltpu.async_copy(
                            xh_h.at[ix.at[p_, i]],
                            rows.at[p_, pl.ds(i * 128, 128)], semG)
                            for i in range(MAC2)])

                    for p_ in (0, 1):
                        for cp in gth[p_]:
                            cp.wait()

                        @pl.loop(0, MAC2 * 8)
                        def _(g):
                            exv = exb[p_, pl.ds(g * 16, 16)]
                            for e in range(16):
                                rr = g * 16 + e
                                rows[p_, rr, :] = rows[p_, rr, :] * exv[e]

                        adds = []
                        for i in range(MAC2):
                            adds.append(pltpu.async_copy(
                                rows.at[p_, pl.ds(i * 128, 128)],
                                acc.at[di.at[p_, i]], sem2, add=True))
                            if len(adds) >= 4:
                                adds.pop(0).wait()
                        for cp in adds:
                            cp.wait()

                plsc.subcore_barrier()
                r0 = sid * ROWS_PER_SUB
                pltpu.sync_copy(acc.at[pl.ds(r0, ROWS_PER_SUB)],
                                out_h.at[pl.ds(r0, ROWS_PER_SUB),
                                         pl.ds(q * 16, 16)])
                plsc.subcore_barrier()

    return k(src2, dst2, xh8, ext3)


def kernel(x, edge_attr, params, edge_index, batch):
    npad = E_PAD - E
    pad_src = jnp.arange(npad, dtype=jnp.int32) % N
    pad_dst = N + 104 + (jnp.arange(npad, dtype=jnp.int32) % 64)
    src2 = jnp.concatenate(
        [edge_index[0].astype(jnp.int32), pad_src]).reshape(R_TOT, 128)
    dst2 = jnp.concatenate(
        [edge_index[1].astype(jnp.int32), pad_dst]).reshape(R_TOT, 128)
    batch3 = batch.astype(jnp.int32).reshape(N // NB, 1, NB)

    p = params
    row = lambda v: v.reshape(1, -1)

    egw = p['eg_W'].reshape(LAYERS, HIDDEN // 2, HEADS, C)
    V = jnp.einsum('lkhc,lhc->lkh', egw, p['att_edge'])
    V = jnp.pad(V, ((0, 0), (0, 0), (0, 16 - HEADS)))

    eye4 = jnp.eye(HEADS, dtype=_f32)
    aM = jnp.einsum('lhc,hg->lchg', p['att_src'], eye4).reshape(
        LAYERS, HIDDEN, HEADS)
    aM = jnp.pad(aM, ((0, 0), (0, 0), (0, 16 - HEADS)))
    dM = jnp.einsum('lhc,hg->lchg', p['att_dst'], eye4).reshape(
        LAYERS, HIDDEN, HEADS)
    dM = jnp.pad(dM, ((0, 0), (0, 0), (0, 16 - HEADS)))

    R4 = jnp.kron(eye4, jnp.ones((1, C), _f32))

    h = _node_enc(x, p['ne_W'], row(p['ne_b']), row(p['ne_g']),
                  row(p['ne_beta']))
    ae = _edge_enc(edge_attr, p['ee_W'], row(p['ee_b']), V)

    xh, asrc16, adst16 = _layer_proj(h, p['gat_W'][0], aM[0], dM[0])
    for l in range(LAYERS):
        ae_pk = ae[l].reshape(E_PAD // 8, 128)
        ext3, denp = _sc_pass1(src2, dst2, asrc16, adst16, ae_pk)
        xh8 = xh.reshape(2 * HEADS * N, 16)
        out = _sc_pass2(src2, dst2, xh8, ext3)
        if l + 1 < LAYERS:
            h, xh, asrc16, adst16 = _fused_norm_proj(
                h, out, denp, R4, row(p['gat_b'][l]), row(p['ln_g'][l]),
                row(p['ln_b'][l]), p['gat_W'][l + 1], aM[l + 1], dM[l + 1])
        else:
            h = _layer_norm_res(h, out, denp, R4, row(p['gat_b'][l]),
                                row(p['ln_g'][l]), row(p['ln_b'][l]))
        g = _pool(h, batch3)
    out = _head_mlp(g, p['f1_W'], row(p['f1_b']), p['f2_W'], row(p['f2_b']),
                    p['c1_W'], row(p['c1_b']), row(p['c2_W'][:, 0]),
                    p['c2_b'].reshape(1, 1))
    return out.reshape(NGRAPH)
